# Initial kernel scaffold; baseline (speedup 1.0000x reference)
import jax
import jax.numpy as jnp
from jax.experimental import pallas as pl


def kernel(Z, block_emb, edge_emb, W1, b1, W2, b2, w_noise, B, edge_index, edge_types):
    raise NotImplementedError("write your pallas kernel here")


# speedup vs baseline: 1.9194x; 1.9194x over previous
import functools

import jax
import jax.numpy as jnp
from jax import lax
from jax.experimental import pallas as pl
from jax.experimental.pallas import tpu as pltpu
from jax.experimental.pallas import tpu_sc as plsc

N = 10000
E = 320000
D = 128
ES = 16
RBF = 16
NLAYERS = 3
NBLK = 100
CUTOFF = 7.0

NC = 2
NS = 16
NW = NC * NS
EPW = E // NW
CH = 80
NCHE = EPW // CH
NP = 10240
RPT = NP // NS
VD = D // 16

_mesh = plsc.VectorSubcoreMesh(core_axis_name="c", subcore_axis_name="s")
_sc_params = pltpu.CompilerParams(needs_layout_passes=False)


@functools.partial(
    pl.kernel,
    mesh=_mesh,
    compiler_params=_sc_params,
    out_type=[
        jax.ShapeDtypeStruct((E,), jnp.float32),
        jax.ShapeDtypeStruct((E,), jnp.float32),
        jax.ShapeDtypeStruct((E,), jnp.float32),
        jax.ShapeDtypeStruct((E,), jnp.float32),
    ],
    scratch_types=[
        pltpu.VMEM((N,), jnp.float32),
        pltpu.VMEM((N,), jnp.float32),
        pltpu.VMEM((N,), jnp.float32),
        pltpu.VMEM((CH,), jnp.int32),
        pltpu.VMEM((CH,), jnp.int32),
        pltpu.VMEM((CH,), jnp.float32),
        pltpu.VMEM((CH,), jnp.float32),
        pltpu.VMEM((CH,), jnp.float32),
        pltpu.VMEM((CH,), jnp.float32),
    ],
)
def _sc_prep(zx, zy, zz, srch, dsth, dx_out, dy_out, dz_out, d2_out,
             xv, yv, zv, sv, dv, dxv, dyv, dzv, d2v):
    cid = lax.axis_index("c")
    sid = lax.axis_index("s")
    wid = cid * NS + sid
    pltpu.sync_copy(zx, xv)
    pltpu.sync_copy(zy, yv)
    pltpu.sync_copy(zz, zv)
    base0 = wid * EPW

    def chunk(k, carry):
        base = base0 + k * CH
        pltpu.sync_copy(srch.at[pl.ds(base, CH)], sv)
        pltpu.sync_copy(dsth.at[pl.ds(base, CH)], dv)

        def grp(g, c2):
            si = sv[pl.ds(g * 16, 16)]
            di = dv[pl.ds(g * 16, 16)]
            dx = plsc.load_gather(xv, [di]) - plsc.load_gather(xv, [si])
            dy = plsc.load_gather(yv, [di]) - plsc.load_gather(yv, [si])
            dz = plsc.load_gather(zv, [di]) - plsc.load_gather(zv, [si])
            dxv[pl.ds(g * 16, 16)] = dx
            dyv[pl.ds(g * 16, 16)] = dy
            dzv[pl.ds(g * 16, 16)] = dz
            d2v[pl.ds(g * 16, 16)] = dx * dx + dy * dy + dz * dz
            return c2

        lax.fori_loop(0, CH // 16, grp, 0)
        pltpu.sync_copy(dxv, dx_out.at[pl.ds(base, CH)])
        pltpu.sync_copy(dyv, dy_out.at[pl.ds(base, CH)])
        pltpu.sync_copy(dzv, dz_out.at[pl.ds(base, CH)])
        pltpu.sync_copy(d2v, d2_out.at[pl.ds(base, CH)])
        return carry

    lax.fori_loop(0, NCHE, chunk, 0)


@functools.partial(
    pl.kernel,
    mesh=_mesh,
    compiler_params=_sc_params,
    out_type=[jax.ShapeDtypeStruct((NC, NP, D), jnp.float32)],
    scratch_types=[
        pltpu.VMEM((CH,), jnp.int32),
        pltpu.VMEM((CH,), jnp.int32),
        pltpu.VMEM((CH, D), jnp.float32),
        pltpu.VMEM((CH, D), jnp.float32),
        pltpu.VMEM((CH, D), jnp.float32),
        pltpu.VMEM_SHARED((NP, D), jnp.float32),
        pltpu.SemaphoreType.DMA,
    ],
)
def _sc_edge(hw, chbm, srch, dsth, agg_out, sv, dv, gv, cv, mv, aggsh, sem):
    cid = lax.axis_index("c")
    sid = lax.axis_index("s")
    wid = cid * NS + sid
    zero16 = jnp.zeros((16,), jnp.float32)

    def zfill(j, carry):
        for t in range(VD):
            gv[j, pl.ds(t * 16, 16)] = zero16
        return carry

    lax.fori_loop(0, CH, zfill, 0)
    for i in range(RPT // CH):
        pltpu.sync_copy(gv, aggsh.at[pl.ds(sid * RPT + i * CH, CH)])
    plsc.subcore_barrier()

    base0 = wid * EPW

    def chunk(k, carry):
        base = base0 + k * CH
        pltpu.sync_copy(srch.at[pl.ds(base, CH)], sv)
        pltpu.sync_copy(dsth.at[pl.ds(base, CH)], dv)
        pltpu.async_copy(hw.at[sv], gv, sem).wait()
        pltpu.sync_copy(chbm.at[pl.ds(base, CH)], cv)

        def row(j, c2):
            for t in range(VD):
                x = gv[j, pl.ds(t * 16, 16)] + cv[j, pl.ds(t * 16, 16)]
                mv[j, pl.ds(t * 16, 16)] = x / (1.0 + jnp.exp(-x))
            return c2

        lax.fori_loop(0, CH, row, 0)
        pltpu.sync_copy(mv, aggsh.at[dv], add=True)
        return carry

    lax.fori_loop(0, NCHE, chunk, 0)
    plsc.subcore_barrier()
    pltpu.sync_copy(aggsh.at[pl.ds(sid * RPT, RPT)],
                    agg_out.at[cid, pl.ds(sid * RPT, RPT)])


@functools.partial(
    pl.kernel,
    mesh=_mesh,
    compiler_params=_sc_params,
    out_type=[jax.ShapeDtypeStruct((NW, 3 * N), jnp.float32)],
    scratch_types=[
        pltpu.VMEM((CH,), jnp.int32),
        pltpu.VMEM((CH,), jnp.int32),
        pltpu.VMEM((CH, D), jnp.float32),
        pltpu.VMEM((CH, D), jnp.float32),
        pltpu.VMEM((128,), jnp.float32),
        pltpu.VMEM((3, CH), jnp.float32),
        pltpu.VMEM((3 * N,), jnp.float32),
        pltpu.SemaphoreType.DMA,
    ],
)
def _sc_noise(hw, chbm, srch, dsth, wnh, dxh, dyh, dzh, nz_out,
              sv, dv, gv, cv, wv, dxyzv, noise3, sem):
    cid = lax.axis_index("c")
    sid = lax.axis_index("s")
    wid = cid * NS + sid
    zero16 = jnp.zeros((16,), jnp.float32)

    def zfill(i, carry):
        noise3[pl.ds(i * 16, 16)] = zero16
        return carry

    lax.fori_loop(0, 3 * N // 16, zfill, 0)
    pltpu.sync_copy(wnh, wv)

    lane = lax.broadcasted_iota(jnp.int32, (16,), 0)
    msk3 = lane < 3
    lane3 = jnp.minimum(lane, 2)
    base0 = wid * EPW

    def chunk(k, carry):
        base = base0 + k * CH
        pltpu.sync_copy(srch.at[pl.ds(base, CH)], sv)
        pltpu.sync_copy(dsth.at[pl.ds(base, CH)], dv)
        pltpu.async_copy(hw.at[sv], gv, sem).wait()
        pltpu.sync_copy(chbm.at[pl.ds(base, CH)], cv)
        pltpu.sync_copy(dxh.at[pl.ds(base, CH)], dxyzv.at[0])
        pltpu.sync_copy(dyh.at[pl.ds(base, CH)], dxyzv.at[1])
        pltpu.sync_copy(dzh.at[pl.ds(base, CH)], dxyzv.at[2])

        def grp(g, c2):
            dgrp = dv[pl.ds(g * 16, 16)]
            for u in range(16):
                j = g * 16 + u
                acc = zero16
                for t in range(VD):
                    x = gv[j, pl.ds(t * 16, 16)] + cv[j, pl.ds(t * 16, 16)]
                    acc = acc + (x / (1.0 + jnp.exp(-x))) * wv[pl.ds(t * 16, 16)]
                sdot = jnp.sum(acc)
                dvec = plsc.load_gather(dxyzv, [lane3, lane * 0 + j],
                                        mask=msk3)
                plsc.addupdate_scatter(noise3, [lane3 * N + dgrp[u]],
                                       dvec * sdot, mask=msk3)
            return c2

        lax.fori_loop(0, CH // 16, grp, 0)
        return carry

    lax.fori_loop(0, NCHE, chunk, 0)
    pltpu.sync_copy(noise3, nz_out.at[wid])


_EB = 2560
_EG = E // _EB
_NB = 1000
_NG = N // _NB


def _c_body(d2_ref, typ_ref, w1r_ref, w1e_ref, eemb_ref, b1_ref, c_ref):
    d2 = d2_ref[0]
    dist = jnp.sqrt(d2 + 1e-8)
    cen = lax.broadcasted_iota(jnp.int32, (RBF, _EB), 0).astype(jnp.float32) * (
        CUTOFF / (RBF - 1))
    rbf_t = jnp.exp(-(dist - cen) ** 2)
    t = typ_ref[0]
    oh_t = (lax.broadcasted_iota(jnp.int32, (4, _EB), 0) == t).astype(jnp.float32)
    ew = jnp.dot(eemb_ref[:], w1e_ref[:],
                 preferred_element_type=jnp.float32)
    c = lax.dot_general(rbf_t, w1r_ref[:], (((0,), (0,)), ((), ())),
                        preferred_element_type=jnp.float32)
    c += lax.dot_general(oh_t, ew, (((0,), (0,)), ((), ())),
                         preferred_element_type=jnp.float32)
    c_ref[:] = c + b1_ref[:]


_c_kernel = pl.pallas_call(
    _c_body,
    grid=(_EG,),
    in_specs=[
        pl.BlockSpec((1, 1, _EB), lambda i: (i, 0, 0)),
        pl.BlockSpec((1, 1, _EB), lambda i: (i, 0, 0)),
        pl.BlockSpec((RBF, D), lambda i: (0, 0)),
        pl.BlockSpec((ES, D), lambda i: (0, 0)),
        pl.BlockSpec((4, ES), lambda i: (0, 0)),
        pl.BlockSpec((1, D), lambda i: (0, 0)),
    ],
    out_specs=pl.BlockSpec((_EB, D), lambda i: (i, 0)),
    out_shape=jax.ShapeDtypeStruct((E, D), jnp.float32),
)


def _embed_body(b_ref, emb_ref, w1h_ref, h_ref, hw_ref):
    b = b_ref[0]
    oh_t = (lax.broadcasted_iota(jnp.int32, (NBLK, _NB), 0) == b).astype(jnp.float32)
    h = lax.dot_general(oh_t, emb_ref[:], (((0,), (0,)), ((), ())),
                        preferred_element_type=jnp.float32)
    h_ref[:] = h
    hw_ref[:] = jnp.dot(h, w1h_ref[:], preferred_element_type=jnp.float32)


_embed_kernel = pl.pallas_call(
    _embed_body,
    grid=(_NG,),
    in_specs=[
        pl.BlockSpec((1, 1, _NB), lambda i: (i, 0, 0)),
        pl.BlockSpec((NBLK, D), lambda i: (0, 0)),
        pl.BlockSpec((D, D), lambda i: (0, 0)),
    ],
    out_specs=[
        pl.BlockSpec((_NB, D), lambda i: (i, 0)),
        pl.BlockSpec((_NB, D), lambda i: (i, 0)),
    ],
    out_shape=[
        jax.ShapeDtypeStruct((N, D), jnp.float32),
        jax.ShapeDtypeStruct((N, D), jnp.float32),
    ],
)


def _make_update(with_next):
    def body(h_ref, aggp_ref, w2_ref, b2_ref, *rest):
        agg = aggp_ref[0] + aggp_ref[1]
        u = jnp.dot(agg, w2_ref[:], preferred_element_type=jnp.float32)
        u = u + b2_ref[:]
        hn = h_ref[:] + u * (1.0 / (1.0 + jnp.exp(-u)))
        if with_next:
            w1n_ref, hn_ref, hw_ref = rest
            hn_ref[:] = hn
            hw_ref[:] = jnp.dot(hn, w1n_ref[:],
                                preferred_element_type=jnp.float32)
        else:
            (hn_ref,) = rest
            hn_ref[:] = hn

    in_specs = [
        pl.BlockSpec((_NB, D), lambda i: (i, 0)),
        pl.BlockSpec((NC, _NB, D), lambda i: (0, i, 0)),
        pl.BlockSpec((D, D), lambda i: (0, 0)),
        pl.BlockSpec((1, D), lambda i: (0, 0)),
    ]
    out_specs = [pl.BlockSpec((_NB, D), lambda i: (i, 0))]
    out_shape = [jax.ShapeDtypeStruct((N, D), jnp.float32)]
    if with_next:
        in_specs.append(pl.BlockSpec((D, D), lambda i: (0, 0)))
        out_specs.append(pl.BlockSpec((_NB, D), lambda i: (i, 0)))
        out_shape.append(jax.ShapeDtypeStruct((N, D), jnp.float32))
    return pl.pallas_call(body, grid=(_NG,), in_specs=in_specs,
                          out_specs=out_specs, out_shape=out_shape)


_update_next = _make_update(True)
_update_final = _make_update(False)

def _nsum_body(p_ref, o_ref):
    o_ref[:] = jnp.sum(p_ref[:], axis=0, keepdims=True)


_noise_sum = pl.pallas_call(
    _nsum_body,
    grid=(1,),
    in_specs=[pl.BlockSpec((NW, 3 * N), lambda i: (0, 0))],
    out_specs=pl.BlockSpec((1, 3 * N), lambda i: (0, 0)),
    out_shape=jax.ShapeDtypeStruct((1, 3 * N), jnp.float32),
)


def kernel(Z, block_emb, edge_emb, W1, b1, W2, b2, w_noise, B, edge_index,
           edge_types):
    f32 = jnp.float32
    zc = Z[:, 0, :].astype(f32)
    src = edge_index[0].astype(jnp.int32)
    dst = edge_index[1].astype(jnp.int32)
    typ = edge_types.astype(jnp.int32)

    dx, dy, dz, d2 = _sc_prep(zc[:, 0], zc[:, 1], zc[:, 2], src, dst)
    d2r = d2.reshape(_EG, 1, _EB)
    typr = typ.reshape(_EG, 1, _EB)
    br = B.astype(jnp.int32).reshape(_NG, 1, _NB)

    h, hw = _embed_kernel(br, block_emb, W1[0, :D])
    wn = w_noise[:, 0]
    for l in range(NLAYERS):
        c = _c_kernel(d2r, typr, W1[l, D + ES:], W1[l, D:D + ES], edge_emb,
                      b1[l:l + 1])
        (aggp,) = _sc_edge(hw, c, src, dst)
        if l < NLAYERS - 1:
            h, hw = _update_next(h, aggp, W2[l], b2[l:l + 1], W1[l + 1, :D])
        else:
            (nzp,) = _sc_noise(hw, c, src, dst, wn, dx, dy, dz)
            (h,) = _update_final(h, aggp, W2[l], b2[l:l + 1])

    noise = _noise_sum(nzp)[0].reshape(3, N).T
    return h, noise


# speedup vs baseline: 4.3521x; 2.2674x over previous
import functools

import jax
import jax.numpy as jnp
from jax import lax
from jax.experimental import pallas as pl
from jax.experimental.pallas import tpu as pltpu
from jax.experimental.pallas import tpu_sc as plsc

N = 10000
E = 320000
D = 128
ES = 16
RBF = 16
NLAYERS = 3
NBLK = 100
CUTOFF = 7.0

NC = 2
NS = 16
NW = NC * NS
EPW = E // NW
CH = 80
NCHE = EPW // CH
NP = 10240
RPT = NP // NS
VD = D // 16

_mesh = plsc.VectorSubcoreMesh(core_axis_name="c", subcore_axis_name="s")
_sc_params = pltpu.CompilerParams(needs_layout_passes=False)


@functools.partial(
    pl.kernel,
    mesh=_mesh,
    compiler_params=_sc_params,
    out_type=[
        jax.ShapeDtypeStruct((E,), jnp.float32),
        jax.ShapeDtypeStruct((E,), jnp.float32),
        jax.ShapeDtypeStruct((E,), jnp.float32),
        jax.ShapeDtypeStruct((E,), jnp.float32),
    ],
    scratch_types=[
        pltpu.VMEM((N,), jnp.float32),
        pltpu.VMEM((N,), jnp.float32),
        pltpu.VMEM((N,), jnp.float32),
        pltpu.VMEM((CH,), jnp.int32),
        pltpu.VMEM((CH,), jnp.int32),
        pltpu.VMEM((CH,), jnp.float32),
        pltpu.VMEM((CH,), jnp.float32),
        pltpu.VMEM((CH,), jnp.float32),
        pltpu.VMEM((CH,), jnp.float32),
    ],
)
def _sc_prep(zx, zy, zz, srch, dsth, dx_out, dy_out, dz_out, d2_out,
             xv, yv, zv, sv, dv, dxv, dyv, dzv, d2v):
    cid = lax.axis_index("c")
    sid = lax.axis_index("s")
    wid = cid * NS + sid
    pltpu.sync_copy(zx, xv)
    pltpu.sync_copy(zy, yv)
    pltpu.sync_copy(zz, zv)
    base0 = wid * EPW

    def chunk(k, carry):
        base = base0 + k * CH
        pltpu.sync_copy(srch.at[pl.ds(base, CH)], sv)
        pltpu.sync_copy(dsth.at[pl.ds(base, CH)], dv)

        def grp(g, c2):
            si = sv[pl.ds(g * 16, 16)]
            di = dv[pl.ds(g * 16, 16)]
            dx = plsc.load_gather(xv, [di]) - plsc.load_gather(xv, [si])
            dy = plsc.load_gather(yv, [di]) - plsc.load_gather(yv, [si])
            dz = plsc.load_gather(zv, [di]) - plsc.load_gather(zv, [si])
            dxv[pl.ds(g * 16, 16)] = dx
            dyv[pl.ds(g * 16, 16)] = dy
            dzv[pl.ds(g * 16, 16)] = dz
            d2v[pl.ds(g * 16, 16)] = dx * dx + dy * dy + dz * dz
            return c2

        lax.fori_loop(0, CH // 16, grp, 0)
        pltpu.sync_copy(dxv, dx_out.at[pl.ds(base, CH)])
        pltpu.sync_copy(dyv, dy_out.at[pl.ds(base, CH)])
        pltpu.sync_copy(dzv, dz_out.at[pl.ds(base, CH)])
        pltpu.sync_copy(d2v, d2_out.at[pl.ds(base, CH)])
        return carry

    lax.fori_loop(0, NCHE, chunk, 0)


def _make_edge(write_m):
    ch = 40
    nch = EPW // ch
    pairs = nch // 2
    out_type = [jax.ShapeDtypeStruct((NC, NP, D), jnp.float32)]
    if write_m:
        out_type.append(jax.ShapeDtypeStruct((E, D), jnp.float32))
    scratch = [
        pltpu.VMEM((ch,), jnp.int32),
        pltpu.VMEM((ch,), jnp.int32),
        pltpu.VMEM((ch,), jnp.int32),
        pltpu.VMEM((ch,), jnp.int32),
        pltpu.VMEM((ch, D), jnp.float32),
        pltpu.VMEM((ch, D), jnp.float32),
        pltpu.VMEM((ch, D), jnp.float32),
        pltpu.VMEM((ch, D), jnp.float32),
        pltpu.VMEM((ch, D), jnp.float32),
        pltpu.VMEM((ch, D), jnp.float32),
        pltpu.VMEM_SHARED((NP, D), jnp.float32),
        pltpu.SemaphoreType.DMA,
        pltpu.SemaphoreType.DMA,
        pltpu.SemaphoreType.DMA,
    ]

    def body(hw, chbm, srch, dsth, *rest):
        if write_m:
            (agg_out, m_out, sv0, dv0, sv1, dv1, gv0, gv1, cv0, cv1,
             mv0, mv1, aggsh, sem0, sem1, semw) = rest
        else:
            (agg_out, sv0, dv0, sv1, dv1, gv0, gv1, cv0, cv1,
             mv0, mv1, aggsh, sem0, sem1, semw) = rest
            m_out = None
        del rest
        cid = lax.axis_index("c")
        sid = lax.axis_index("s")
        wid = cid * NS + sid
        zero16 = jnp.zeros((16,), jnp.float32)
        base0 = wid * EPW

        def zfill(j, carry):
            for t in range(VD):
                gv0[j, pl.ds(t * 16, 16)] = zero16
            return carry

        lax.fori_loop(0, ch, zfill, 0)
        for i in range(RPT // ch):
            pltpu.sync_copy(gv0, aggsh.at[pl.ds(sid * RPT + i * ch, ch)])
        plsc.subcore_barrier()

        def issue(k, sv, dv, gv, cv, sem):
            base = base0 + k * ch
            pltpu.sync_copy(srch.at[pl.ds(base, ch)], sv)
            pltpu.sync_copy(dsth.at[pl.ds(base, ch)], dv)
            pltpu.async_copy(hw.at[sv], gv, sem)
            pltpu.async_copy(chbm.at[pl.ds(base, ch)], cv, sem)

        def wait(sv, gv, cv, sem):
            pltpu.make_async_copy(hw.at[sv], gv, sem).wait()
            pltpu.make_async_copy(chbm.at[pl.ds(0, ch)], cv, sem).wait()

        def process(k, dv, gv, cv, mv, first):
            if write_m and not first:
                pltpu.make_async_copy(mv, m_out.at[pl.ds(0, ch)], semw).wait()

            def row(j, c2):
                for t in range(VD):
                    x = gv[j, pl.ds(t * 16, 16)] + cv[j, pl.ds(t * 16, 16)]
                    mv[j, pl.ds(t * 16, 16)] = x / (1.0 + jnp.exp(-x))
                return c2

            lax.fori_loop(0, ch, row, 0)
            if write_m:
                pltpu.async_copy(mv, m_out.at[pl.ds(base0 + k * ch, ch)],
                                 semw)
            pltpu.sync_copy(mv, aggsh.at[dv], add=True)

        issue(0, sv0, dv0, gv0, cv0, sem0)
        issue(1, sv1, dv1, gv1, cv1, sem1)
        wait(sv0, gv0, cv0, sem0)
        process(0, dv0, gv0, cv0, mv0, True)
        issue(2, sv0, dv0, gv0, cv0, sem0)
        wait(sv1, gv1, cv1, sem1)
        process(1, dv1, gv1, cv1, mv1, True)

        def pipe(i, carry):
            issue(2 * i + 3, sv1, dv1, gv1, cv1, sem1)
            wait(sv0, gv0, cv0, sem0)
            process(2 * i + 2, dv0, gv0, cv0, mv0, False)
            issue(2 * i + 4, sv0, dv0, gv0, cv0, sem0)
            wait(sv1, gv1, cv1, sem1)
            process(2 * i + 3, dv1, gv1, cv1, mv1, False)
            return carry

        lax.fori_loop(0, pairs - 2, pipe, 0)
        issue(nch - 1, sv1, dv1, gv1, cv1, sem1)
        wait(sv0, gv0, cv0, sem0)
        process(nch - 2, dv0, gv0, cv0, mv0, False)
        wait(sv1, gv1, cv1, sem1)
        process(nch - 1, dv1, gv1, cv1, mv1, False)
        if write_m:
            pltpu.make_async_copy(mv0, m_out.at[pl.ds(0, ch)], semw).wait()
            pltpu.make_async_copy(mv1, m_out.at[pl.ds(0, ch)], semw).wait()

        plsc.subcore_barrier()
        pltpu.sync_copy(aggsh.at[pl.ds(sid * RPT, RPT)],
                        agg_out.at[cid, pl.ds(sid * RPT, RPT)])

    return pl.kernel(body, mesh=_mesh, out_type=out_type,
                     scratch_types=scratch, compiler_params=_sc_params)


_sc_edge = _make_edge(write_m=False)
_sc_edge_m = _make_edge(write_m=True)


_CH2 = 2000
_NCH2 = EPW // _CH2


@functools.partial(
    pl.kernel,
    mesh=_mesh,
    compiler_params=_sc_params,
    out_type=[jax.ShapeDtypeStruct((NW, 3 * N), jnp.float32)],
    scratch_types=[
        pltpu.VMEM((_CH2,), jnp.int32),
        pltpu.VMEM((_CH2,), jnp.float32),
        pltpu.VMEM((_CH2,), jnp.float32),
        pltpu.VMEM((_CH2,), jnp.float32),
        pltpu.VMEM((3 * N,), jnp.float32),
    ],
)
def _sc_nscatter(nxh, nyh, nzh, dsth, nz_out, dv, xv, yv, zv, noise3):
    cid = lax.axis_index("c")
    sid = lax.axis_index("s")
    wid = cid * NS + sid
    zero16 = jnp.zeros((16,), jnp.float32)

    def zfill(i, carry):
        noise3[pl.ds(i * 16, 16)] = zero16
        return carry

    lax.fori_loop(0, 3 * N // 16, zfill, 0)

    lane = lax.broadcasted_iota(jnp.int32, (16,), 0)
    msk3 = lane < 3
    lane3 = jnp.minimum(lane, 2)
    base0 = wid * EPW

    def chunk(k, carry):
        base = base0 + k * _CH2
        pltpu.sync_copy(dsth.at[pl.ds(base, _CH2)], dv)
        pltpu.sync_copy(nxh.at[pl.ds(base, _CH2)], xv)
        pltpu.sync_copy(nyh.at[pl.ds(base, _CH2)], yv)
        pltpu.sync_copy(nzh.at[pl.ds(base, _CH2)], zv)

        def grp(g, c2):
            dgrp = dv[pl.ds(g * 16, 16)]
            xg = xv[pl.ds(g * 16, 16)]
            yg = yv[pl.ds(g * 16, 16)]
            zg = zv[pl.ds(g * 16, 16)]
            for u in range(16):
                val = jnp.where(lane == 0, xg[u],
                                jnp.where(lane == 1, yg[u], zg[u]))
                plsc.addupdate_scatter(noise3, [lane3 * N + dgrp[u]], val,
                                       mask=msk3)
            return c2

        lax.fori_loop(0, _CH2 // 16, grp, 0)
        return carry

    lax.fori_loop(0, _NCH2, chunk, 0)
    pltpu.sync_copy(noise3, nz_out.at[wid])


_EB = 2560
_EG = E // _EB
_NB = 1000
_NG = N // _NB


def _c_body(d2_ref, typ_ref, w1r_ref, w1e_ref, eemb_ref, b1_ref, c_ref):
    d2 = d2_ref[0]
    dist = jnp.sqrt(d2 + 1e-8)
    cen = lax.broadcasted_iota(jnp.int32, (RBF, _EB), 0).astype(jnp.float32) * (
        CUTOFF / (RBF - 1))
    rbf_t = jnp.exp(-(dist - cen) ** 2)
    t = typ_ref[0]
    oh_t = (lax.broadcasted_iota(jnp.int32, (4, _EB), 0) == t).astype(jnp.float32)
    ew = jnp.dot(eemb_ref[:], w1e_ref[:],
                 preferred_element_type=jnp.float32)
    c = lax.dot_general(rbf_t, w1r_ref[:], (((0,), (0,)), ((), ())),
                        preferred_element_type=jnp.float32)
    c += lax.dot_general(oh_t, ew, (((0,), (0,)), ((), ())),
                         preferred_element_type=jnp.float32)
    c_ref[:] = c + b1_ref[:]


_c_kernel = pl.pallas_call(
    _c_body,
    grid=(_EG,),
    in_specs=[
        pl.BlockSpec((1, 1, _EB), lambda i: (i, 0, 0)),
        pl.BlockSpec((1, 1, _EB), lambda i: (i, 0, 0)),
        pl.BlockSpec((RBF, D), lambda i: (0, 0)),
        pl.BlockSpec((ES, D), lambda i: (0, 0)),
        pl.BlockSpec((4, ES), lambda i: (0, 0)),
        pl.BlockSpec((1, D), lambda i: (0, 0)),
    ],
    out_specs=pl.BlockSpec((_EB, D), lambda i: (i, 0)),
    out_shape=jax.ShapeDtypeStruct((E, D), jnp.float32),
)


def _embed_body(b_ref, emb_ref, w1h_ref, h_ref, hw_ref):
    b = b_ref[0]
    oh_t = (lax.broadcasted_iota(jnp.int32, (NBLK, _NB), 0) == b).astype(jnp.float32)
    h = lax.dot_general(oh_t, emb_ref[:], (((0,), (0,)), ((), ())),
                        preferred_element_type=jnp.float32)
    h_ref[:] = h
    hw_ref[:] = jnp.dot(h, w1h_ref[:], preferred_element_type=jnp.float32)


_embed_kernel = pl.pallas_call(
    _embed_body,
    grid=(_NG,),
    in_specs=[
        pl.BlockSpec((1, 1, _NB), lambda i: (i, 0, 0)),
        pl.BlockSpec((NBLK, D), lambda i: (0, 0)),
        pl.BlockSpec((D, D), lambda i: (0, 0)),
    ],
    out_specs=[
        pl.BlockSpec((_NB, D), lambda i: (i, 0)),
        pl.BlockSpec((_NB, D), lambda i: (i, 0)),
    ],
    out_shape=[
        jax.ShapeDtypeStruct((N, D), jnp.float32),
        jax.ShapeDtypeStruct((N, D), jnp.float32),
    ],
)


def _make_update(with_next):
    def body(h_ref, aggp_ref, w2_ref, b2_ref, *rest):
        agg = aggp_ref[0] + aggp_ref[1]
        u = jnp.dot(agg, w2_ref[:], preferred_element_type=jnp.float32)
        u = u + b2_ref[:]
        hn = h_ref[:] + u * (1.0 / (1.0 + jnp.exp(-u)))
        if with_next:
            w1n_ref, hn_ref, hw_ref = rest
            hn_ref[:] = hn
            hw_ref[:] = jnp.dot(hn, w1n_ref[:],
                                preferred_element_type=jnp.float32)
        else:
            (hn_ref,) = rest
            hn_ref[:] = hn

    in_specs = [
        pl.BlockSpec((_NB, D), lambda i: (i, 0)),
        pl.BlockSpec((NC, _NB, D), lambda i: (0, i, 0)),
        pl.BlockSpec((D, D), lambda i: (0, 0)),
        pl.BlockSpec((1, D), lambda i: (0, 0)),
    ]
    out_specs = [pl.BlockSpec((_NB, D), lambda i: (i, 0))]
    out_shape = [jax.ShapeDtypeStruct((N, D), jnp.float32)]
    if with_next:
        in_specs.append(pl.BlockSpec((D, D), lambda i: (0, 0)))
        out_specs.append(pl.BlockSpec((_NB, D), lambda i: (i, 0)))
        out_shape.append(jax.ShapeDtypeStruct((N, D), jnp.float32))
    return pl.pallas_call(body, grid=(_NG,), in_specs=in_specs,
                          out_specs=out_specs, out_shape=out_shape)


_update_next = _make_update(True)
_update_final = _make_update(False)


def _nw_body(w_ref, m_ref, dx_ref, dy_ref, dz_ref, ox_ref, oy_ref, oz_ref):
    s = lax.dot_general(w_ref[:], m_ref[:], (((1,), (1,)), ((), ())),
                        preferred_element_type=jnp.float32)
    ox_ref[0] = dx_ref[0] * s
    oy_ref[0] = dy_ref[0] * s
    oz_ref[0] = dz_ref[0] * s


_noise_w = pl.pallas_call(
    _nw_body,
    grid=(_EG,),
    in_specs=[
        pl.BlockSpec((1, D), lambda i: (0, 0)),
        pl.BlockSpec((_EB, D), lambda i: (i, 0)),
        pl.BlockSpec((1, 1, _EB), lambda i: (i, 0, 0)),
        pl.BlockSpec((1, 1, _EB), lambda i: (i, 0, 0)),
        pl.BlockSpec((1, 1, _EB), lambda i: (i, 0, 0)),
    ],
    out_specs=[
        pl.BlockSpec((1, 1, _EB), lambda i: (i, 0, 0)),
        pl.BlockSpec((1, 1, _EB), lambda i: (i, 0, 0)),
        pl.BlockSpec((1, 1, _EB), lambda i: (i, 0, 0)),
    ],
    out_shape=[
        jax.ShapeDtypeStruct((_EG, 1, _EB), jnp.float32),
        jax.ShapeDtypeStruct((_EG, 1, _EB), jnp.float32),
        jax.ShapeDtypeStruct((_EG, 1, _EB), jnp.float32),
    ],
)


def _nsum_body(p_ref, o_ref):
    o_ref[:] = jnp.sum(p_ref[:], axis=0, keepdims=True)


_noise_sum = pl.pallas_call(
    _nsum_body,
    grid=(1,),
    in_specs=[pl.BlockSpec((NW, 3 * N), lambda i: (0, 0))],
    out_specs=pl.BlockSpec((1, 3 * N), lambda i: (0, 0)),
    out_shape=jax.ShapeDtypeStruct((1, 3 * N), jnp.float32),
)


def kernel(Z, block_emb, edge_emb, W1, b1, W2, b2, w_noise, B, edge_index,
           edge_types):
    f32 = jnp.float32
    zc = Z[:, 0, :].astype(f32)
    src = edge_index[0].astype(jnp.int32)
    dst = edge_index[1].astype(jnp.int32)
    typ = edge_types.astype(jnp.int32)

    dx, dy, dz, d2 = _sc_prep(zc[:, 0], zc[:, 1], zc[:, 2], src, dst)
    d2r = d2.reshape(_EG, 1, _EB)
    typr = typ.reshape(_EG, 1, _EB)
    br = B.astype(jnp.int32).reshape(_NG, 1, _NB)

    h, hw = _embed_kernel(br, block_emb, W1[0, :D])
    for l in range(NLAYERS):
        c = _c_kernel(d2r, typr, W1[l, D + ES:], W1[l, D:D + ES], edge_emb,
                      b1[l:l + 1])
        if l < NLAYERS - 1:
            (aggp,) = _sc_edge(hw, c, src, dst)
            h, hw = _update_next(h, aggp, W2[l], b2[l:l + 1], W1[l + 1, :D])
        else:
            aggp, m = _sc_edge_m(hw, c, src, dst)
            ox, oy, oz = _noise_w(w_noise.T,
                                  m,
                                  dx.reshape(_EG, 1, _EB),
                                  dy.reshape(_EG, 1, _EB),
                                  dz.reshape(_EG, 1, _EB))
            (nzp,) = _sc_nscatter(ox.reshape(E), oy.reshape(E),
                                  oz.reshape(E), dst)
            (h,) = _update_final(h, aggp, W2[l], b2[l:l + 1])

    noise = _noise_sum(nzp)[0].reshape(3, N).T
    return h, noise


# speedup vs baseline: 4.5505x; 1.0456x over previous
import functools

import jax
import jax.numpy as jnp
from jax import lax
from jax.experimental import pallas as pl
from jax.experimental.pallas import tpu as pltpu
from jax.experimental.pallas import tpu_sc as plsc

N = 10000
E = 320000
D = 128
ES = 16
RBF = 16
NLAYERS = 3
NBLK = 100
CUTOFF = 7.0

NC = 2
NS = 16
NW = NC * NS
EPW = E // NW
PCH = 400
NPCH = EPW // PCH
NP = 10240
RPT = NP // NS
VD = D // 16

_mesh = plsc.VectorSubcoreMesh(core_axis_name="c", subcore_axis_name="s")
_sc_params = pltpu.CompilerParams(needs_layout_passes=False)


@functools.partial(
    pl.kernel,
    mesh=_mesh,
    compiler_params=_sc_params,
    out_type=[
        jax.ShapeDtypeStruct((E,), jnp.float32),
        jax.ShapeDtypeStruct((E,), jnp.float32),
        jax.ShapeDtypeStruct((E,), jnp.float32),
        jax.ShapeDtypeStruct((E,), jnp.float32),
    ],
    scratch_types=[
        pltpu.VMEM((N,), jnp.float32),
        pltpu.VMEM((N,), jnp.float32),
        pltpu.VMEM((N,), jnp.float32),
        pltpu.VMEM((PCH,), jnp.int32),
        pltpu.VMEM((PCH,), jnp.int32),
        pltpu.VMEM((PCH,), jnp.float32),
        pltpu.VMEM((PCH,), jnp.float32),
        pltpu.VMEM((PCH,), jnp.float32),
        pltpu.VMEM((PCH,), jnp.float32),
    ],
)
def _sc_prep(zx, zy, zz, srch, dsth, dx_out, dy_out, dz_out, d2_out,
             xv, yv, zv, sv, dv, dxv, dyv, dzv, d2v):
    cid = lax.axis_index("c")
    sid = lax.axis_index("s")
    wid = cid * NS + sid
    pltpu.sync_copy(zx, xv)
    pltpu.sync_copy(zy, yv)
    pltpu.sync_copy(zz, zv)
    base0 = wid * EPW

    def chunk(k, carry):
        base = base0 + k * PCH
        pltpu.sync_copy(srch.at[pl.ds(base, PCH)], sv)
        pltpu.sync_copy(dsth.at[pl.ds(base, PCH)], dv)

        def grp(g, c2):
            si = sv[pl.ds(g * 16, 16)]
            di = dv[pl.ds(g * 16, 16)]
            dx = plsc.load_gather(xv, [di]) - plsc.load_gather(xv, [si])
            dy = plsc.load_gather(yv, [di]) - plsc.load_gather(yv, [si])
            dz = plsc.load_gather(zv, [di]) - plsc.load_gather(zv, [si])
            dxv[pl.ds(g * 16, 16)] = dx
            dyv[pl.ds(g * 16, 16)] = dy
            dzv[pl.ds(g * 16, 16)] = dz
            d2v[pl.ds(g * 16, 16)] = dx * dx + dy * dy + dz * dz
            return c2

        lax.fori_loop(0, PCH // 16, grp, 0)
        pltpu.sync_copy(dxv, dx_out.at[pl.ds(base, PCH)])
        pltpu.sync_copy(dyv, dy_out.at[pl.ds(base, PCH)])
        pltpu.sync_copy(dzv, dz_out.at[pl.ds(base, PCH)])
        pltpu.sync_copy(d2v, d2_out.at[pl.ds(base, PCH)])
        return carry

    lax.fori_loop(0, NPCH, chunk, 0)


def _make_edge(write_m):
    ch = 40
    nch = EPW // ch
    pairs = nch // 2
    out_type = [jax.ShapeDtypeStruct((NC, NP, D), jnp.float32)]
    if write_m:
        out_type.append(jax.ShapeDtypeStruct((E, D), jnp.float32))
    scratch = [
        pltpu.VMEM((ch,), jnp.int32),
        pltpu.VMEM((ch,), jnp.int32),
        pltpu.VMEM((ch,), jnp.int32),
        pltpu.VMEM((ch,), jnp.int32),
        pltpu.VMEM((ch, D), jnp.float32),
        pltpu.VMEM((ch, D), jnp.float32),
        pltpu.VMEM((ch, D), jnp.float32),
        pltpu.VMEM((ch, D), jnp.float32),
        pltpu.VMEM_SHARED((NP, D), jnp.float32),
        pltpu.SemaphoreType.DMA,
        pltpu.SemaphoreType.DMA,
    ]
    if write_m:
        scratch += [
            pltpu.VMEM((ch, D), jnp.float32),
            pltpu.VMEM((ch, D), jnp.float32),
            pltpu.SemaphoreType.DMA,
        ]

    def body(hw, chbm, srch, dsth, *rest):
        if write_m:
            (agg_out, m_out, sv0, dv0, sv1, dv1, gv0, gv1, cv0, cv1,
             aggsh, sem0, sem1, mv0, mv1, semw) = rest
        else:
            (agg_out, sv0, dv0, sv1, dv1, gv0, gv1, cv0, cv1,
             aggsh, sem0, sem1) = rest
            mv0 = gv0
            mv1 = gv1
            m_out = semw = None
        del rest
        cid = lax.axis_index("c")
        sid = lax.axis_index("s")
        wid = cid * NS + sid
        zero16 = jnp.zeros((16,), jnp.float32)
        base0 = wid * EPW

        def zfill(j, carry):
            for t in range(VD):
                gv0[j, pl.ds(t * 16, 16)] = zero16
            return carry

        lax.fori_loop(0, ch, zfill, 0)
        for i in range(RPT // ch):
            pltpu.sync_copy(gv0, aggsh.at[pl.ds(sid * RPT + i * ch, ch)])
        plsc.subcore_barrier()

        def issue(k, sv, dv, gv, cv, sem):
            base = base0 + k * ch
            pltpu.sync_copy(srch.at[pl.ds(base, ch)], sv)
            pltpu.sync_copy(dsth.at[pl.ds(base, ch)], dv)
            pltpu.async_copy(hw.at[sv], gv, sem)
            pltpu.async_copy(chbm.at[pl.ds(base, ch)], cv, sem)

        def wait(sv, gv, cv, sem):
            pltpu.make_async_copy(hw.at[sv], gv, sem).wait()
            pltpu.make_async_copy(chbm.at[pl.ds(0, ch)], cv, sem).wait()

        def process(k, dv, gv, cv, mv, first):
            if write_m and not first:
                pltpu.make_async_copy(mv, m_out.at[pl.ds(0, ch)], semw).wait()

            def row(j, c2):
                for u in range(2):
                    r = 2 * j + u
                    for t in range(VD):
                        x = gv[r, pl.ds(t * 16, 16)] + cv[r, pl.ds(t * 16, 16)]
                        mv[r, pl.ds(t * 16, 16)] = x / (1.0 + jnp.exp(-x))
                return c2

            lax.fori_loop(0, ch // 2, row, 0)
            if write_m:
                pltpu.async_copy(mv, m_out.at[pl.ds(base0 + k * ch, ch)],
                                 semw)
            pltpu.sync_copy(mv, aggsh.at[dv], add=True)

        issue(0, sv0, dv0, gv0, cv0, sem0)
        issue(1, sv1, dv1, gv1, cv1, sem1)
        wait(sv0, gv0, cv0, sem0)
        process(0, dv0, gv0, cv0, mv0, True)
        issue(2, sv0, dv0, gv0, cv0, sem0)
        wait(sv1, gv1, cv1, sem1)
        process(1, dv1, gv1, cv1, mv1, True)

        def pipe(i, carry):
            issue(2 * i + 3, sv1, dv1, gv1, cv1, sem1)
            wait(sv0, gv0, cv0, sem0)
            process(2 * i + 2, dv0, gv0, cv0, mv0, False)
            issue(2 * i + 4, sv0, dv0, gv0, cv0, sem0)
            wait(sv1, gv1, cv1, sem1)
            process(2 * i + 3, dv1, gv1, cv1, mv1, False)
            return carry

        lax.fori_loop(0, pairs - 2, pipe, 0)
        issue(nch - 1, sv1, dv1, gv1, cv1, sem1)
        wait(sv0, gv0, cv0, sem0)
        process(nch - 2, dv0, gv0, cv0, mv0, False)
        wait(sv1, gv1, cv1, sem1)
        process(nch - 1, dv1, gv1, cv1, mv1, False)
        if write_m:
            pltpu.make_async_copy(mv0, m_out.at[pl.ds(0, ch)], semw).wait()
            pltpu.make_async_copy(mv1, m_out.at[pl.ds(0, ch)], semw).wait()

        plsc.subcore_barrier()
        pltpu.sync_copy(aggsh.at[pl.ds(sid * RPT, RPT)],
                        agg_out.at[cid, pl.ds(sid * RPT, RPT)])

    return pl.kernel(body, mesh=_mesh, out_type=out_type,
                     scratch_types=scratch, compiler_params=_sc_params)


_sc_edge = _make_edge(write_m=False)
_sc_edge_m = _make_edge(write_m=True)


_CH2 = 2000
_NCH2 = EPW // _CH2


@functools.partial(
    pl.kernel,
    mesh=_mesh,
    compiler_params=_sc_params,
    out_type=[jax.ShapeDtypeStruct((NW, 3 * N), jnp.float32)],
    scratch_types=[
        pltpu.VMEM((_CH2,), jnp.int32),
        pltpu.VMEM((_CH2,), jnp.float32),
        pltpu.VMEM((_CH2,), jnp.float32),
        pltpu.VMEM((_CH2,), jnp.float32),
        pltpu.VMEM((3 * N,), jnp.float32),
    ],
)
def _sc_nscatter(nxh, nyh, nzh, dsth, nz_out, dv, xv, yv, zv, noise3):
    cid = lax.axis_index("c")
    sid = lax.axis_index("s")
    wid = cid * NS + sid
    zero16 = jnp.zeros((16,), jnp.float32)

    def zfill(i, carry):
        noise3[pl.ds(i * 16, 16)] = zero16
        return carry

    lax.fori_loop(0, 3 * N // 16, zfill, 0)

    lane = lax.broadcasted_iota(jnp.int32, (16,), 0)
    msk3 = lane < 3
    lane3 = jnp.minimum(lane, 2)
    base0 = wid * EPW

    def chunk(k, carry):
        base = base0 + k * _CH2
        pltpu.sync_copy(dsth.at[pl.ds(base, _CH2)], dv)
        pltpu.sync_copy(nxh.at[pl.ds(base, _CH2)], xv)
        pltpu.sync_copy(nyh.at[pl.ds(base, _CH2)], yv)
        pltpu.sync_copy(nzh.at[pl.ds(base, _CH2)], zv)

        def grp(g, c2):
            dgrp = dv[pl.ds(g * 16, 16)]
            xg = xv[pl.ds(g * 16, 16)]
            yg = yv[pl.ds(g * 16, 16)]
            zg = zv[pl.ds(g * 16, 16)]
            for u in range(16):
                val = jnp.where(lane == 0, xg[u],
                                jnp.where(lane == 1, yg[u], zg[u]))
                plsc.addupdate_scatter(noise3, [lane3 * N + dgrp[u]], val,
                                       mask=msk3)
            return c2

        lax.fori_loop(0, _CH2 // 16, grp, 0)
        return carry

    lax.fori_loop(0, _NCH2, chunk, 0)
    pltpu.sync_copy(noise3, nz_out.at[wid])


_EB = 2560
_EG = E // _EB
_NB = 1000
_NG = N // _NB


def _c_body(d2_ref, typ_ref, w1r_ref, w1e_ref, eemb_ref, b1_ref, c_ref):
    d2 = d2_ref[0]
    dist = jnp.sqrt(d2 + 1e-8)
    cen = lax.broadcasted_iota(jnp.int32, (RBF, _EB), 0).astype(jnp.float32) * (
        CUTOFF / (RBF - 1))
    rbf_t = jnp.exp(-(dist - cen) ** 2)
    t = typ_ref[0]
    oh_t = (lax.broadcasted_iota(jnp.int32, (4, _EB), 0) == t).astype(jnp.float32)
    ew = jnp.dot(eemb_ref[:], w1e_ref[:],
                 preferred_element_type=jnp.float32)
    c = lax.dot_general(rbf_t, w1r_ref[:], (((0,), (0,)), ((), ())),
                        preferred_element_type=jnp.float32)
    c += lax.dot_general(oh_t, ew, (((0,), (0,)), ((), ())),
                         preferred_element_type=jnp.float32)
    c_ref[:] = c + b1_ref[:]


_c_kernel = pl.pallas_call(
    _c_body,
    grid=(_EG,),
    in_specs=[
        pl.BlockSpec((1, 1, _EB), lambda i: (i, 0, 0)),
        pl.BlockSpec((1, 1, _EB), lambda i: (i, 0, 0)),
        pl.BlockSpec((RBF, D), lambda i: (0, 0)),
        pl.BlockSpec((ES, D), lambda i: (0, 0)),
        pl.BlockSpec((4, ES), lambda i: (0, 0)),
        pl.BlockSpec((1, D), lambda i: (0, 0)),
    ],
    out_specs=pl.BlockSpec((_EB, D), lambda i: (i, 0)),
    out_shape=jax.ShapeDtypeStruct((E, D), jnp.float32),
)


def _embed_body(b_ref, emb_ref, w1h_ref, h_ref, hw_ref):
    b = b_ref[0]
    oh_t = (lax.broadcasted_iota(jnp.int32, (NBLK, _NB), 0) == b).astype(jnp.float32)
    h = lax.dot_general(oh_t, emb_ref[:], (((0,), (0,)), ((), ())),
                        preferred_element_type=jnp.float32)
    h_ref[:] = h
    hw_ref[:] = jnp.dot(h, w1h_ref[:], preferred_element_type=jnp.float32)


_embed_kernel = pl.pallas_call(
    _embed_body,
    grid=(_NG,),
    in_specs=[
        pl.BlockSpec((1, 1, _NB), lambda i: (i, 0, 0)),
        pl.BlockSpec((NBLK, D), lambda i: (0, 0)),
        pl.BlockSpec((D, D), lambda i: (0, 0)),
    ],
    out_specs=[
        pl.BlockSpec((_NB, D), lambda i: (i, 0)),
        pl.BlockSpec((_NB, D), lambda i: (i, 0)),
    ],
    out_shape=[
        jax.ShapeDtypeStruct((N, D), jnp.float32),
        jax.ShapeDtypeStruct((N, D), jnp.float32),
    ],
)


def _make_update(with_next):
    def body(h_ref, aggp_ref, w2_ref, b2_ref, *rest):
        agg = aggp_ref[0] + aggp_ref[1]
        u = jnp.dot(agg, w2_ref[:], preferred_element_type=jnp.float32)
        u = u + b2_ref[:]
        hn = h_ref[:] + u * (1.0 / (1.0 + jnp.exp(-u)))
        if with_next:
            w1n_ref, hn_ref, hw_ref = rest
            hn_ref[:] = hn
            hw_ref[:] = jnp.dot(hn, w1n_ref[:],
                                preferred_element_type=jnp.float32)
        else:
            (hn_ref,) = rest
            hn_ref[:] = hn

    in_specs = [
        pl.BlockSpec((_NB, D), lambda i: (i, 0)),
        pl.BlockSpec((NC, _NB, D), lambda i: (0, i, 0)),
        pl.BlockSpec((D, D), lambda i: (0, 0)),
        pl.BlockSpec((1, D), lambda i: (0, 0)),
    ]
    out_specs = [pl.BlockSpec((_NB, D), lambda i: (i, 0))]
    out_shape = [jax.ShapeDtypeStruct((N, D), jnp.float32)]
    if with_next:
        in_specs.append(pl.BlockSpec((D, D), lambda i: (0, 0)))
        out_specs.append(pl.BlockSpec((_NB, D), lambda i: (i, 0)))
        out_shape.append(jax.ShapeDtypeStruct((N, D), jnp.float32))
    return pl.pallas_call(body, grid=(_NG,), in_specs=in_specs,
                          out_specs=out_specs, out_shape=out_shape)


_update_next = _make_update(True)
_update_final = _make_update(False)


def _nw_body(w_ref, m_ref, dx_ref, dy_ref, dz_ref, ox_ref, oy_ref, oz_ref):
    s = lax.dot_general(w_ref[:], m_ref[:], (((1,), (1,)), ((), ())),
                        preferred_element_type=jnp.float32)
    ox_ref[0] = dx_ref[0] * s
    oy_ref[0] = dy_ref[0] * s
    oz_ref[0] = dz_ref[0] * s


_noise_w = pl.pallas_call(
    _nw_body,
    grid=(_EG,),
    in_specs=[
        pl.BlockSpec((1, D), lambda i: (0, 0)),
        pl.BlockSpec((_EB, D), lambda i: (i, 0)),
        pl.BlockSpec((1, 1, _EB), lambda i: (i, 0, 0)),
        pl.BlockSpec((1, 1, _EB), lambda i: (i, 0, 0)),
        pl.BlockSpec((1, 1, _EB), lambda i: (i, 0, 0)),
    ],
    out_specs=[
        pl.BlockSpec((1, 1, _EB), lambda i: (i, 0, 0)),
        pl.BlockSpec((1, 1, _EB), lambda i: (i, 0, 0)),
        pl.BlockSpec((1, 1, _EB), lambda i: (i, 0, 0)),
    ],
    out_shape=[
        jax.ShapeDtypeStruct((_EG, 1, _EB), jnp.float32),
        jax.ShapeDtypeStruct((_EG, 1, _EB), jnp.float32),
        jax.ShapeDtypeStruct((_EG, 1, _EB), jnp.float32),
    ],
)


def _nsum_body(p_ref, o_ref):
    o_ref[:] = jnp.sum(p_ref[:], axis=0, keepdims=True)


_noise_sum = pl.pallas_call(
    _nsum_body,
    grid=(1,),
    in_specs=[pl.BlockSpec((NW, 3 * N), lambda i: (0, 0))],
    out_specs=pl.BlockSpec((1, 3 * N), lambda i: (0, 0)),
    out_shape=jax.ShapeDtypeStruct((1, 3 * N), jnp.float32),
)


def kernel(Z, block_emb, edge_emb, W1, b1, W2, b2, w_noise, B, edge_index,
           edge_types):
    f32 = jnp.float32
    zc = Z[:, 0, :].astype(f32)
    src = edge_index[0].astype(jnp.int32)
    dst = edge_index[1].astype(jnp.int32)
    typ = edge_types.astype(jnp.int32)

    dx, dy, dz, d2 = _sc_prep(zc[:, 0], zc[:, 1], zc[:, 2], src, dst)
    d2r = d2.reshape(_EG, 1, _EB)
    typr = typ.reshape(_EG, 1, _EB)
    br = B.astype(jnp.int32).reshape(_NG, 1, _NB)

    h, hw = _embed_kernel(br, block_emb, W1[0, :D])
    for l in range(NLAYERS):
        c = _c_kernel(d2r, typr, W1[l, D + ES:], W1[l, D:D + ES], edge_emb,
                      b1[l:l + 1])
        if l < NLAYERS - 1:
            (aggp,) = _sc_edge(hw, c, src, dst)
            h, hw = _update_next(h, aggp, W2[l], b2[l:l + 1], W1[l + 1, :D])
        else:
            aggp, m = _sc_edge_m(hw, c, src, dst)
            ox, oy, oz = _noise_w(w_noise.T,
                                  m,
                                  dx.reshape(_EG, 1, _EB),
                                  dy.reshape(_EG, 1, _EB),
                                  dz.reshape(_EG, 1, _EB))
            (nzp,) = _sc_nscatter(ox.reshape(E), oy.reshape(E),
                                  oz.reshape(E), dst)
            (h,) = _update_final(h, aggp, W2[l], b2[l:l + 1])

    noise = _noise_sum(nzp)[0].reshape(3, N).T
    return h, noise


# speedup vs baseline: 5.2290x; 1.1491x over previous
import functools

import jax
import jax.numpy as jnp
from jax import lax
from jax.experimental import pallas as pl
from jax.experimental.pallas import tpu as pltpu
from jax.experimental.pallas import tpu_sc as plsc

N = 10000
E = 320000
D = 128
ES = 16
RBF = 16
NLAYERS = 3
NBLK = 100
CUTOFF = 7.0

NC = 2
NS = 16
NW = NC * NS
EPW = E // NW
PCH = 400
NPCH = EPW // PCH
NP = 10240
RPT = NP // NS
VD = D // 16

_mesh = plsc.VectorSubcoreMesh(core_axis_name="c", subcore_axis_name="s")
_sc_params = pltpu.CompilerParams(needs_layout_passes=False)


@functools.partial(
    pl.kernel,
    mesh=_mesh,
    compiler_params=_sc_params,
    out_type=[
        jax.ShapeDtypeStruct((E,), jnp.float32),
        jax.ShapeDtypeStruct((E,), jnp.float32),
        jax.ShapeDtypeStruct((E,), jnp.float32),
        jax.ShapeDtypeStruct((E,), jnp.float32),
    ],
    scratch_types=[
        pltpu.VMEM((N,), jnp.float32),
        pltpu.VMEM((N,), jnp.float32),
        pltpu.VMEM((N,), jnp.float32),
        pltpu.VMEM((PCH,), jnp.int32),
        pltpu.VMEM((PCH,), jnp.int32),
        pltpu.VMEM((PCH,), jnp.float32),
        pltpu.VMEM((PCH,), jnp.float32),
        pltpu.VMEM((PCH,), jnp.float32),
        pltpu.VMEM((PCH,), jnp.float32),
    ],
)
def _sc_prep(zx, zy, zz, srch, dsth, dx_out, dy_out, dz_out, d2_out,
             xv, yv, zv, sv, dv, dxv, dyv, dzv, d2v):
    cid = lax.axis_index("c")
    sid = lax.axis_index("s")
    wid = cid * NS + sid
    pltpu.sync_copy(zx, xv)
    pltpu.sync_copy(zy, yv)
    pltpu.sync_copy(zz, zv)
    base0 = wid * EPW

    def chunk(k, carry):
        base = base0 + k * PCH
        pltpu.sync_copy(srch.at[pl.ds(base, PCH)], sv)
        pltpu.sync_copy(dsth.at[pl.ds(base, PCH)], dv)

        def grp(g, c2):
            si = sv[pl.ds(g * 16, 16)]
            di = dv[pl.ds(g * 16, 16)]
            dx = plsc.load_gather(xv, [di]) - plsc.load_gather(xv, [si])
            dy = plsc.load_gather(yv, [di]) - plsc.load_gather(yv, [si])
            dz = plsc.load_gather(zv, [di]) - plsc.load_gather(zv, [si])
            dxv[pl.ds(g * 16, 16)] = dx
            dyv[pl.ds(g * 16, 16)] = dy
            dzv[pl.ds(g * 16, 16)] = dz
            d2v[pl.ds(g * 16, 16)] = dx * dx + dy * dy + dz * dz
            return c2

        lax.fori_loop(0, PCH // 16, grp, 0)
        pltpu.sync_copy(dxv, dx_out.at[pl.ds(base, PCH)])
        pltpu.sync_copy(dyv, dy_out.at[pl.ds(base, PCH)])
        pltpu.sync_copy(dzv, dz_out.at[pl.ds(base, PCH)])
        pltpu.sync_copy(d2v, d2_out.at[pl.ds(base, PCH)])
        return carry

    lax.fori_loop(0, NPCH, chunk, 0)


def _make_edge(write_m):
    ch = 40 if write_m else 80
    nch = EPW // ch
    pairs = nch // 2
    out_type = [jax.ShapeDtypeStruct((NC, NP, D), jnp.float32)]
    if write_m:
        out_type.append(jax.ShapeDtypeStruct((E, D), jnp.float32))
    scratch = [
        pltpu.VMEM((ch,), jnp.int32),
        pltpu.VMEM((ch,), jnp.int32),
        pltpu.VMEM((ch,), jnp.int32),
        pltpu.VMEM((ch,), jnp.int32),
        pltpu.VMEM((ch, D), jnp.float32),
        pltpu.VMEM((ch, D), jnp.float32),
        pltpu.VMEM((ch, D), jnp.float32),
        pltpu.VMEM((ch, D), jnp.float32),
        pltpu.VMEM_SHARED((NP, D), jnp.float32),
        pltpu.SemaphoreType.DMA,
        pltpu.SemaphoreType.DMA,
    ]
    if write_m:
        scratch += [
            pltpu.VMEM((ch, D), jnp.float32),
            pltpu.VMEM((ch, D), jnp.float32),
            pltpu.SemaphoreType.DMA,
        ]

    def body(hw, chbm, srch, dsth, *rest):
        if write_m:
            (agg_out, m_out, sv0, dv0, sv1, dv1, gv0, gv1, cv0, cv1,
             aggsh, sem0, sem1, mv0, mv1, semw) = rest
        else:
            (agg_out, sv0, dv0, sv1, dv1, gv0, gv1, cv0, cv1,
             aggsh, sem0, sem1) = rest
            mv0 = gv0
            mv1 = gv1
            m_out = semw = None
        del rest
        cid = lax.axis_index("c")
        sid = lax.axis_index("s")
        wid = cid * NS + sid
        zero16 = jnp.zeros((16,), jnp.float32)
        base0 = wid * EPW

        def zfill(j, carry):
            for t in range(VD):
                gv0[j, pl.ds(t * 16, 16)] = zero16
            return carry

        lax.fori_loop(0, ch, zfill, 0)
        for i in range(RPT // ch):
            pltpu.sync_copy(gv0, aggsh.at[pl.ds(sid * RPT + i * ch, ch)])
        plsc.subcore_barrier()

        def issue(k, sv, dv, gv, cv, sem):
            base = base0 + k * ch
            pltpu.sync_copy(srch.at[pl.ds(base, ch)], sv)
            pltpu.sync_copy(dsth.at[pl.ds(base, ch)], dv)
            pltpu.async_copy(hw.at[sv], gv, sem)
            pltpu.async_copy(chbm.at[pl.ds(base, ch)], cv, sem)

        def wait(sv, gv, cv, sem):
            pltpu.make_async_copy(hw.at[sv], gv, sem).wait()
            pltpu.make_async_copy(chbm.at[pl.ds(0, ch)], cv, sem).wait()

        def process(k, dv, gv, cv, mv, first):
            if write_m and not first:
                pltpu.make_async_copy(mv, m_out.at[pl.ds(0, ch)], semw).wait()

            def row(j, c2):
                for u in range(2):
                    r = 2 * j + u
                    for t in range(VD):
                        x = gv[r, pl.ds(t * 16, 16)] + cv[r, pl.ds(t * 16, 16)]
                        mv[r, pl.ds(t * 16, 16)] = x / (1.0 + jnp.exp(-x))
                return c2

            lax.fori_loop(0, ch // 2, row, 0)
            if write_m:
                pltpu.async_copy(mv, m_out.at[pl.ds(base0 + k * ch, ch)],
                                 semw)
            pltpu.sync_copy(mv, aggsh.at[dv], add=True)

        issue(0, sv0, dv0, gv0, cv0, sem0)
        issue(1, sv1, dv1, gv1, cv1, sem1)
        wait(sv0, gv0, cv0, sem0)
        process(0, dv0, gv0, cv0, mv0, True)
        issue(2, sv0, dv0, gv0, cv0, sem0)
        wait(sv1, gv1, cv1, sem1)
        process(1, dv1, gv1, cv1, mv1, True)

        def pipe(i, carry):
            issue(2 * i + 3, sv1, dv1, gv1, cv1, sem1)
            wait(sv0, gv0, cv0, sem0)
            process(2 * i + 2, dv0, gv0, cv0, mv0, False)
            issue(2 * i + 4, sv0, dv0, gv0, cv0, sem0)
            wait(sv1, gv1, cv1, sem1)
            process(2 * i + 3, dv1, gv1, cv1, mv1, False)
            return carry

        if nch % 2 == 0:
            lax.fori_loop(0, pairs - 2, pipe, 0)
            issue(nch - 1, sv1, dv1, gv1, cv1, sem1)
            wait(sv0, gv0, cv0, sem0)
            process(nch - 2, dv0, gv0, cv0, mv0, False)
            wait(sv1, gv1, cv1, sem1)
            process(nch - 1, dv1, gv1, cv1, mv1, False)
        else:
            lax.fori_loop(0, (nch - 3) // 2, pipe, 0)
            wait(sv0, gv0, cv0, sem0)
            process(nch - 1, dv0, gv0, cv0, mv0, False)
        if write_m:
            pltpu.make_async_copy(mv0, m_out.at[pl.ds(0, ch)], semw).wait()
            pltpu.make_async_copy(mv1, m_out.at[pl.ds(0, ch)], semw).wait()

        plsc.subcore_barrier()
        pltpu.sync_copy(aggsh.at[pl.ds(sid * RPT, RPT)],
                        agg_out.at[cid, pl.ds(sid * RPT, RPT)])

    return pl.kernel(body, mesh=_mesh, out_type=out_type,
                     scratch_types=scratch, compiler_params=_sc_params)


_sc_edge = _make_edge(write_m=False)
_sc_edge_m = _make_edge(write_m=True)


_CH2 = 2000
_NCH2 = EPW // _CH2


@functools.partial(
    pl.kernel,
    mesh=_mesh,
    compiler_params=_sc_params,
    out_type=[jax.ShapeDtypeStruct((NW, 3 * N), jnp.float32)],
    scratch_types=[
        pltpu.VMEM((_CH2,), jnp.int32),
        pltpu.VMEM((_CH2,), jnp.float32),
        pltpu.VMEM((_CH2,), jnp.float32),
        pltpu.VMEM((_CH2,), jnp.float32),
        pltpu.VMEM((3 * N,), jnp.float32),
    ],
)
def _sc_nscatter(nxh, nyh, nzh, dsth, nz_out, dv, xv, yv, zv, noise3):
    cid = lax.axis_index("c")
    sid = lax.axis_index("s")
    wid = cid * NS + sid
    zero16 = jnp.zeros((16,), jnp.float32)

    def zfill(i, carry):
        noise3[pl.ds(i * 16, 16)] = zero16
        return carry

    lax.fori_loop(0, 3 * N // 16, zfill, 0)

    lane = lax.broadcasted_iota(jnp.int32, (16,), 0)
    msk3 = lane < 3
    lane3 = jnp.minimum(lane, 2)
    base0 = wid * EPW

    def chunk(k, carry):
        base = base0 + k * _CH2
        pltpu.sync_copy(dsth.at[pl.ds(base, _CH2)], dv)
        pltpu.sync_copy(nxh.at[pl.ds(base, _CH2)], xv)
        pltpu.sync_copy(nyh.at[pl.ds(base, _CH2)], yv)
        pltpu.sync_copy(nzh.at[pl.ds(base, _CH2)], zv)

        def grp(g, c2):
            dgrp = dv[pl.ds(g * 16, 16)]
            xg = xv[pl.ds(g * 16, 16)]
            yg = yv[pl.ds(g * 16, 16)]
            zg = zv[pl.ds(g * 16, 16)]
            for u in range(16):
                val = jnp.where(lane == 0, xg[u],
                                jnp.where(lane == 1, yg[u], zg[u]))
                plsc.addupdate_scatter(noise3, [lane3 * N + dgrp[u]], val,
                                       mask=msk3)
            return c2

        lax.fori_loop(0, _CH2 // 16, grp, 0)
        return carry

    lax.fori_loop(0, _NCH2, chunk, 0)
    pltpu.sync_copy(noise3, nz_out.at[wid])


_EB = 2560
_EG = E // _EB
_NB = 1000
_NG = N // _NB


def _c_body(d2_ref, typ_ref, w1r_ref, w1e_ref, eemb_ref, b1_ref, c_ref):
    d2 = d2_ref[0]
    dist = jnp.sqrt(d2 + 1e-8)
    cen = lax.broadcasted_iota(jnp.int32, (RBF, _EB), 0).astype(jnp.float32) * (
        CUTOFF / (RBF - 1))
    rbf_t = jnp.exp(-(dist - cen) ** 2)
    t = typ_ref[0]
    oh_t = (lax.broadcasted_iota(jnp.int32, (4, _EB), 0) == t).astype(jnp.float32)
    ew = jnp.dot(eemb_ref[:], w1e_ref[:],
                 preferred_element_type=jnp.float32)
    c = lax.dot_general(rbf_t, w1r_ref[:], (((0,), (0,)), ((), ())),
                        preferred_element_type=jnp.float32)
    c += lax.dot_general(oh_t, ew, (((0,), (0,)), ((), ())),
                         preferred_element_type=jnp.float32)
    c_ref[:] = c + b1_ref[:]


_c_kernel = pl.pallas_call(
    _c_body,
    grid=(_EG,),
    in_specs=[
        pl.BlockSpec((1, 1, _EB), lambda i: (i, 0, 0)),
        pl.BlockSpec((1, 1, _EB), lambda i: (i, 0, 0)),
        pl.BlockSpec((RBF, D), lambda i: (0, 0)),
        pl.BlockSpec((ES, D), lambda i: (0, 0)),
        pl.BlockSpec((4, ES), lambda i: (0, 0)),
        pl.BlockSpec((1, D), lambda i: (0, 0)),
    ],
    out_specs=pl.BlockSpec((_EB, D), lambda i: (i, 0)),
    out_shape=jax.ShapeDtypeStruct((E, D), jnp.float32),
)


def _embed_body(b_ref, emb_ref, w1h_ref, h_ref, hw_ref):
    b = b_ref[0]
    oh_t = (lax.broadcasted_iota(jnp.int32, (NBLK, _NB), 0) == b).astype(jnp.float32)
    h = lax.dot_general(oh_t, emb_ref[:], (((0,), (0,)), ((), ())),
                        preferred_element_type=jnp.float32)
    h_ref[:] = h
    hw_ref[:] = jnp.dot(h, w1h_ref[:], preferred_element_type=jnp.float32)


_embed_kernel = pl.pallas_call(
    _embed_body,
    grid=(_NG,),
    in_specs=[
        pl.BlockSpec((1, 1, _NB), lambda i: (i, 0, 0)),
        pl.BlockSpec((NBLK, D), lambda i: (0, 0)),
        pl.BlockSpec((D, D), lambda i: (0, 0)),
    ],
    out_specs=[
        pl.BlockSpec((_NB, D), lambda i: (i, 0)),
        pl.BlockSpec((_NB, D), lambda i: (i, 0)),
    ],
    out_shape=[
        jax.ShapeDtypeStruct((N, D), jnp.float32),
        jax.ShapeDtypeStruct((N, D), jnp.float32),
    ],
)


def _make_update(with_next):
    def body(h_ref, aggp_ref, w2_ref, b2_ref, *rest):
        agg = aggp_ref[0] + aggp_ref[1]
        u = jnp.dot(agg, w2_ref[:], preferred_element_type=jnp.float32)
        u = u + b2_ref[:]
        hn = h_ref[:] + u * (1.0 / (1.0 + jnp.exp(-u)))
        if with_next:
            w1n_ref, hn_ref, hw_ref = rest
            hn_ref[:] = hn
            hw_ref[:] = jnp.dot(hn, w1n_ref[:],
                                preferred_element_type=jnp.float32)
        else:
            (hn_ref,) = rest
            hn_ref[:] = hn

    in_specs = [
        pl.BlockSpec((_NB, D), lambda i: (i, 0)),
        pl.BlockSpec((NC, _NB, D), lambda i: (0, i, 0)),
        pl.BlockSpec((D, D), lambda i: (0, 0)),
        pl.BlockSpec((1, D), lambda i: (0, 0)),
    ]
    out_specs = [pl.BlockSpec((_NB, D), lambda i: (i, 0))]
    out_shape = [jax.ShapeDtypeStruct((N, D), jnp.float32)]
    if with_next:
        in_specs.append(pl.BlockSpec((D, D), lambda i: (0, 0)))
        out_specs.append(pl.BlockSpec((_NB, D), lambda i: (i, 0)))
        out_shape.append(jax.ShapeDtypeStruct((N, D), jnp.float32))
    return pl.pallas_call(body, grid=(_NG,), in_specs=in_specs,
                          out_specs=out_specs, out_shape=out_shape)


_update_next = _make_update(True)
_update_final = _make_update(False)


def _nw_body(w_ref, m_ref, dx_ref, dy_ref, dz_ref, ox_ref, oy_ref, oz_ref):
    s = lax.dot_general(w_ref[:], m_ref[:], (((1,), (1,)), ((), ())),
                        preferred_element_type=jnp.float32)
    ox_ref[0] = dx_ref[0] * s
    oy_ref[0] = dy_ref[0] * s
    oz_ref[0] = dz_ref[0] * s


_noise_w = pl.pallas_call(
    _nw_body,
    grid=(_EG,),
    in_specs=[
        pl.BlockSpec((1, D), lambda i: (0, 0)),
        pl.BlockSpec((_EB, D), lambda i: (i, 0)),
        pl.BlockSpec((1, 1, _EB), lambda i: (i, 0, 0)),
        pl.BlockSpec((1, 1, _EB), lambda i: (i, 0, 0)),
        pl.BlockSpec((1, 1, _EB), lambda i: (i, 0, 0)),
    ],
    out_specs=[
        pl.BlockSpec((1, 1, _EB), lambda i: (i, 0, 0)),
        pl.BlockSpec((1, 1, _EB), lambda i: (i, 0, 0)),
        pl.BlockSpec((1, 1, _EB), lambda i: (i, 0, 0)),
    ],
    out_shape=[
        jax.ShapeDtypeStruct((_EG, 1, _EB), jnp.float32),
        jax.ShapeDtypeStruct((_EG, 1, _EB), jnp.float32),
        jax.ShapeDtypeStruct((_EG, 1, _EB), jnp.float32),
    ],
)


def _nsum_body(p_ref, o_ref):
    o_ref[:] = jnp.sum(p_ref[:], axis=0, keepdims=True)


_noise_sum = pl.pallas_call(
    _nsum_body,
    grid=(1,),
    in_specs=[pl.BlockSpec((NW, 3 * N), lambda i: (0, 0))],
    out_specs=pl.BlockSpec((1, 3 * N), lambda i: (0, 0)),
    out_shape=jax.ShapeDtypeStruct((1, 3 * N), jnp.float32),
)


def kernel(Z, block_emb, edge_emb, W1, b1, W2, b2, w_noise, B, edge_index,
           edge_types):
    f32 = jnp.float32
    zc = Z[:, 0, :].astype(f32)
    src = edge_index[0].astype(jnp.int32)
    dst = edge_index[1].astype(jnp.int32)
    typ = edge_types.astype(jnp.int32)

    dx, dy, dz, d2 = _sc_prep(zc[:, 0], zc[:, 1], zc[:, 2], src, dst)
    d2r = d2.reshape(_EG, 1, _EB)
    typr = typ.reshape(_EG, 1, _EB)
    br = B.astype(jnp.int32).reshape(_NG, 1, _NB)

    h, hw = _embed_kernel(br, block_emb, W1[0, :D])
    for l in range(NLAYERS):
        c = _c_kernel(d2r, typr, W1[l, D + ES:], W1[l, D:D + ES], edge_emb,
                      b1[l:l + 1])
        if l < NLAYERS - 1:
            (aggp,) = _sc_edge(hw, c, src, dst)
            h, hw = _update_next(h, aggp, W2[l], b2[l:l + 1], W1[l + 1, :D])
        else:
            aggp, m = _sc_edge_m(hw, c, src, dst)
            ox, oy, oz = _noise_w(w_noise.T,
                                  m,
                                  dx.reshape(_EG, 1, _EB),
                                  dy.reshape(_EG, 1, _EB),
                                  dz.reshape(_EG, 1, _EB))
            (nzp,) = _sc_nscatter(ox.reshape(E), oy.reshape(E),
                                  oz.reshape(E), dst)
            (h,) = _update_final(h, aggp, W2[l], b2[l:l + 1])

    noise = _noise_sum(nzp)[0].reshape(3, N).T
    return h, noise


# speedup vs baseline: 5.2930x; 1.0122x over previous
import functools

import jax
import jax.numpy as jnp
from jax import lax
from jax.experimental import pallas as pl
from jax.experimental.pallas import tpu as pltpu
from jax.experimental.pallas import tpu_sc as plsc

N = 10000
E = 320000
D = 128
ES = 16
RBF = 16
NLAYERS = 3
NBLK = 100
CUTOFF = 7.0

NC = 2
NS = 16
NW = NC * NS
EPW = E // NW
PCH = 400
NPCH = EPW // PCH
NP = 10240
RPT = NP // NS
VD = D // 16

_mesh = plsc.VectorSubcoreMesh(core_axis_name="c", subcore_axis_name="s")
_sc_params = pltpu.CompilerParams(needs_layout_passes=False)


@functools.partial(
    pl.kernel,
    mesh=_mesh,
    compiler_params=_sc_params,
    out_type=[
        jax.ShapeDtypeStruct((E,), jnp.float32),
        jax.ShapeDtypeStruct((E,), jnp.float32),
        jax.ShapeDtypeStruct((E,), jnp.float32),
        jax.ShapeDtypeStruct((E,), jnp.float32),
    ],
    scratch_types=[
        pltpu.VMEM((N,), jnp.float32),
        pltpu.VMEM((N,), jnp.float32),
        pltpu.VMEM((N,), jnp.float32),
        pltpu.VMEM((PCH,), jnp.int32),
        pltpu.VMEM((PCH,), jnp.int32),
        pltpu.VMEM((PCH,), jnp.float32),
        pltpu.VMEM((PCH,), jnp.float32),
        pltpu.VMEM((PCH,), jnp.float32),
        pltpu.VMEM((PCH,), jnp.float32),
    ],
)
def _sc_prep(zx, zy, zz, srch, dsth, dx_out, dy_out, dz_out, d2_out,
             xv, yv, zv, sv, dv, dxv, dyv, dzv, d2v):
    cid = lax.axis_index("c")
    sid = lax.axis_index("s")
    wid = cid * NS + sid
    pltpu.sync_copy(zx, xv)
    pltpu.sync_copy(zy, yv)
    pltpu.sync_copy(zz, zv)
    base0 = wid * EPW

    def chunk(k, carry):
        base = base0 + k * PCH
        pltpu.sync_copy(srch.at[pl.ds(base, PCH)], sv)
        pltpu.sync_copy(dsth.at[pl.ds(base, PCH)], dv)

        def grp(g, c2):
            si = sv[pl.ds(g * 16, 16)]
            di = dv[pl.ds(g * 16, 16)]
            dx = plsc.load_gather(xv, [di]) - plsc.load_gather(xv, [si])
            dy = plsc.load_gather(yv, [di]) - plsc.load_gather(yv, [si])
            dz = plsc.load_gather(zv, [di]) - plsc.load_gather(zv, [si])
            dxv[pl.ds(g * 16, 16)] = dx
            dyv[pl.ds(g * 16, 16)] = dy
            dzv[pl.ds(g * 16, 16)] = dz
            d2v[pl.ds(g * 16, 16)] = dx * dx + dy * dy + dz * dz
            return c2

        lax.fori_loop(0, PCH // 16, grp, 0)
        pltpu.sync_copy(dxv, dx_out.at[pl.ds(base, PCH)])
        pltpu.sync_copy(dyv, dy_out.at[pl.ds(base, PCH)])
        pltpu.sync_copy(dzv, dz_out.at[pl.ds(base, PCH)])
        pltpu.sync_copy(d2v, d2_out.at[pl.ds(base, PCH)])
        return carry

    lax.fori_loop(0, NPCH, chunk, 0)


def _make_edge(write_m):
    ch = 40 if write_m else 80
    nch = EPW // ch
    pairs = nch // 2
    out_type = [jax.ShapeDtypeStruct((NC, NP, D), jnp.float32)]
    if write_m:
        out_type.append(jax.ShapeDtypeStruct((E, D), jnp.float32))
    scratch = [
        pltpu.VMEM((ch,), jnp.int32),
        pltpu.VMEM((ch,), jnp.int32),
        pltpu.VMEM((ch,), jnp.int32),
        pltpu.VMEM((ch,), jnp.int32),
        pltpu.VMEM((ch, D), jnp.float32),
        pltpu.VMEM((ch, D), jnp.float32),
        pltpu.VMEM((ch, D), jnp.float32),
        pltpu.VMEM((ch, D), jnp.float32),
        pltpu.VMEM_SHARED((NP, D), jnp.float32),
        pltpu.SemaphoreType.DMA,
        pltpu.SemaphoreType.DMA,
    ]
    if write_m:
        scratch += [
            pltpu.VMEM((ch, D), jnp.float32),
            pltpu.VMEM((ch, D), jnp.float32),
            pltpu.SemaphoreType.DMA,
        ]

    def body(hw, chbm, srch, dsth, *rest):
        if write_m:
            (agg_out, m_out, sv0, dv0, sv1, dv1, gv0, gv1, cv0, cv1,
             aggsh, sem0, sem1, mv0, mv1, semw) = rest
        else:
            (agg_out, sv0, dv0, sv1, dv1, gv0, gv1, cv0, cv1,
             aggsh, sem0, sem1) = rest
            mv0 = gv0
            mv1 = gv1
            m_out = semw = None
        del rest
        cid = lax.axis_index("c")
        sid = lax.axis_index("s")
        wid = cid * NS + sid
        zero16 = jnp.zeros((16,), jnp.float32)
        base0 = wid * EPW

        def zfill(j, carry):
            for t in range(VD):
                gv0[j, pl.ds(t * 16, 16)] = zero16
            return carry

        lax.fori_loop(0, ch, zfill, 0)
        for i in range(RPT // ch):
            pltpu.sync_copy(gv0, aggsh.at[pl.ds(sid * RPT + i * ch, ch)])
        plsc.subcore_barrier()

        def issue(k, sv, dv, gv, cv, sem):
            base = base0 + k * ch
            pltpu.sync_copy(srch.at[pl.ds(base, ch)], sv)
            pltpu.sync_copy(dsth.at[pl.ds(base, ch)], dv)
            pltpu.async_copy(hw.at[sv], gv, sem)
            pltpu.async_copy(chbm.at[pl.ds(base, ch)], cv, sem)

        def wait(sv, gv, cv, sem):
            pltpu.make_async_copy(hw.at[sv], gv, sem).wait()
            pltpu.make_async_copy(chbm.at[pl.ds(0, ch)], cv, sem).wait()

        def process(k, dv, gv, cv, mv, first):
            if write_m and not first:
                pltpu.make_async_copy(mv, m_out.at[pl.ds(0, ch)], semw).wait()

            def row(j, c2):
                for u in range(4):
                    r = 4 * j + u
                    for t in range(VD):
                        x = gv[r, pl.ds(t * 16, 16)] + cv[r, pl.ds(t * 16, 16)]
                        mv[r, pl.ds(t * 16, 16)] = x / (1.0 + jnp.exp(-x))
                return c2

            lax.fori_loop(0, ch // 4, row, 0)
            if write_m:
                pltpu.async_copy(mv, m_out.at[pl.ds(base0 + k * ch, ch)],
                                 semw)
            pltpu.sync_copy(mv, aggsh.at[dv], add=True)

        issue(0, sv0, dv0, gv0, cv0, sem0)
        issue(1, sv1, dv1, gv1, cv1, sem1)
        wait(sv0, gv0, cv0, sem0)
        process(0, dv0, gv0, cv0, mv0, True)
        issue(2, sv0, dv0, gv0, cv0, sem0)
        wait(sv1, gv1, cv1, sem1)
        process(1, dv1, gv1, cv1, mv1, True)

        def pipe(i, carry):
            issue(2 * i + 3, sv1, dv1, gv1, cv1, sem1)
            wait(sv0, gv0, cv0, sem0)
            process(2 * i + 2, dv0, gv0, cv0, mv0, False)
            issue(2 * i + 4, sv0, dv0, gv0, cv0, sem0)
            wait(sv1, gv1, cv1, sem1)
            process(2 * i + 3, dv1, gv1, cv1, mv1, False)
            return carry

        if nch % 2 == 0:
            lax.fori_loop(0, pairs - 2, pipe, 0)
            issue(nch - 1, sv1, dv1, gv1, cv1, sem1)
            wait(sv0, gv0, cv0, sem0)
            process(nch - 2, dv0, gv0, cv0, mv0, False)
            wait(sv1, gv1, cv1, sem1)
            process(nch - 1, dv1, gv1, cv1, mv1, False)
        else:
            lax.fori_loop(0, (nch - 3) // 2, pipe, 0)
            wait(sv0, gv0, cv0, sem0)
            process(nch - 1, dv0, gv0, cv0, mv0, False)
        if write_m:
            pltpu.make_async_copy(mv0, m_out.at[pl.ds(0, ch)], semw).wait()
            pltpu.make_async_copy(mv1, m_out.at[pl.ds(0, ch)], semw).wait()

        plsc.subcore_barrier()
        pltpu.sync_copy(aggsh.at[pl.ds(sid * RPT, RPT)],
                        agg_out.at[cid, pl.ds(sid * RPT, RPT)])

    return pl.kernel(body, mesh=_mesh, out_type=out_type,
                     scratch_types=scratch, compiler_params=_sc_params)


_sc_edge = _make_edge(write_m=False)
_sc_edge_m = _make_edge(write_m=True)


_CH2 = 2000
_NCH2 = EPW // _CH2


@functools.partial(
    pl.kernel,
    mesh=_mesh,
    compiler_params=_sc_params,
    out_type=[jax.ShapeDtypeStruct((NW, 3 * N), jnp.float32)],
    scratch_types=[
        pltpu.VMEM((_CH2,), jnp.int32),
        pltpu.VMEM((_CH2,), jnp.float32),
        pltpu.VMEM((_CH2,), jnp.float32),
        pltpu.VMEM((_CH2,), jnp.float32),
        pltpu.VMEM((3 * N,), jnp.float32),
    ],
)
def _sc_nscatter(nxh, nyh, nzh, dsth, nz_out, dv, xv, yv, zv, noise3):
    cid = lax.axis_index("c")
    sid = lax.axis_index("s")
    wid = cid * NS + sid
    zero16 = jnp.zeros((16,), jnp.float32)

    def zfill(i, carry):
        noise3[pl.ds(i * 16, 16)] = zero16
        return carry

    lax.fori_loop(0, 3 * N // 16, zfill, 0)

    lane = lax.broadcasted_iota(jnp.int32, (16,), 0)
    msk3 = lane < 3
    lane3 = jnp.minimum(lane, 2)
    base0 = wid * EPW

    def chunk(k, carry):
        base = base0 + k * _CH2
        pltpu.sync_copy(dsth.at[pl.ds(base, _CH2)], dv)
        pltpu.sync_copy(nxh.at[pl.ds(base, _CH2)], xv)
        pltpu.sync_copy(nyh.at[pl.ds(base, _CH2)], yv)
        pltpu.sync_copy(nzh.at[pl.ds(base, _CH2)], zv)

        def grp(g, c2):
            dgrp = dv[pl.ds(g * 16, 16)]
            xg = xv[pl.ds(g * 16, 16)]
            yg = yv[pl.ds(g * 16, 16)]
            zg = zv[pl.ds(g * 16, 16)]
            for u in range(16):
                val = jnp.where(lane == 0, xg[u],
                                jnp.where(lane == 1, yg[u], zg[u]))
                plsc.addupdate_scatter(noise3, [lane3 * N + dgrp[u]], val,
                                       mask=msk3)
            return c2

        lax.fori_loop(0, _CH2 // 16, grp, 0)
        return carry

    lax.fori_loop(0, _NCH2, chunk, 0)
    pltpu.sync_copy(noise3, nz_out.at[wid])


_EB = 2560
_EG = E // _EB
_NB = 1000
_NG = N // _NB


def _c_body(d2_ref, typ_ref, w1r_ref, w1e_ref, eemb_ref, b1_ref, c_ref):
    d2 = d2_ref[0]
    dist = jnp.sqrt(d2 + 1e-8)
    cen = lax.broadcasted_iota(jnp.int32, (RBF, _EB), 0).astype(jnp.float32) * (
        CUTOFF / (RBF - 1))
    rbf_t = jnp.exp(-(dist - cen) ** 2)
    t = typ_ref[0]
    oh_t = (lax.broadcasted_iota(jnp.int32, (4, _EB), 0) == t).astype(jnp.float32)
    ew = jnp.dot(eemb_ref[:], w1e_ref[:],
                 preferred_element_type=jnp.float32)
    c = lax.dot_general(rbf_t, w1r_ref[:], (((0,), (0,)), ((), ())),
                        preferred_element_type=jnp.float32)
    c += lax.dot_general(oh_t, ew, (((0,), (0,)), ((), ())),
                         preferred_element_type=jnp.float32)
    c_ref[:] = c + b1_ref[:]


_c_kernel = pl.pallas_call(
    _c_body,
    grid=(_EG,),
    in_specs=[
        pl.BlockSpec((1, 1, _EB), lambda i: (i, 0, 0)),
        pl.BlockSpec((1, 1, _EB), lambda i: (i, 0, 0)),
        pl.BlockSpec((RBF, D), lambda i: (0, 0)),
        pl.BlockSpec((ES, D), lambda i: (0, 0)),
        pl.BlockSpec((4, ES), lambda i: (0, 0)),
        pl.BlockSpec((1, D), lambda i: (0, 0)),
    ],
    out_specs=pl.BlockSpec((_EB, D), lambda i: (i, 0)),
    out_shape=jax.ShapeDtypeStruct((E, D), jnp.float32),
)


def _embed_body(b_ref, emb_ref, w1h_ref, h_ref, hw_ref):
    b = b_ref[0]
    oh_t = (lax.broadcasted_iota(jnp.int32, (NBLK, _NB), 0) == b).astype(jnp.float32)
    h = lax.dot_general(oh_t, emb_ref[:], (((0,), (0,)), ((), ())),
                        preferred_element_type=jnp.float32)
    h_ref[:] = h
    hw_ref[:] = jnp.dot(h, w1h_ref[:], preferred_element_type=jnp.float32)


_embed_kernel = pl.pallas_call(
    _embed_body,
    grid=(_NG,),
    in_specs=[
        pl.BlockSpec((1, 1, _NB), lambda i: (i, 0, 0)),
        pl.BlockSpec((NBLK, D), lambda i: (0, 0)),
        pl.BlockSpec((D, D), lambda i: (0, 0)),
    ],
    out_specs=[
        pl.BlockSpec((_NB, D), lambda i: (i, 0)),
        pl.BlockSpec((_NB, D), lambda i: (i, 0)),
    ],
    out_shape=[
        jax.ShapeDtypeStruct((N, D), jnp.float32),
        jax.ShapeDtypeStruct((N, D), jnp.float32),
    ],
)


def _make_update(with_next):
    def body(h_ref, aggp_ref, w2_ref, b2_ref, *rest):
        agg = aggp_ref[0] + aggp_ref[1]
        u = jnp.dot(agg, w2_ref[:], preferred_element_type=jnp.float32)
        u = u + b2_ref[:]
        hn = h_ref[:] + u * (1.0 / (1.0 + jnp.exp(-u)))
        if with_next:
            w1n_ref, hn_ref, hw_ref = rest
            hn_ref[:] = hn
            hw_ref[:] = jnp.dot(hn, w1n_ref[:],
                                preferred_element_type=jnp.float32)
        else:
            (hn_ref,) = rest
            hn_ref[:] = hn

    in_specs = [
        pl.BlockSpec((_NB, D), lambda i: (i, 0)),
        pl.BlockSpec((NC, _NB, D), lambda i: (0, i, 0)),
        pl.BlockSpec((D, D), lambda i: (0, 0)),
        pl.BlockSpec((1, D), lambda i: (0, 0)),
    ]
    out_specs = [pl.BlockSpec((_NB, D), lambda i: (i, 0))]
    out_shape = [jax.ShapeDtypeStruct((N, D), jnp.float32)]
    if with_next:
        in_specs.append(pl.BlockSpec((D, D), lambda i: (0, 0)))
        out_specs.append(pl.BlockSpec((_NB, D), lambda i: (i, 0)))
        out_shape.append(jax.ShapeDtypeStruct((N, D), jnp.float32))
    return pl.pallas_call(body, grid=(_NG,), in_specs=in_specs,
                          out_specs=out_specs, out_shape=out_shape)


_update_next = _make_update(True)
_update_final = _make_update(False)


def _nw_body(w_ref, m_ref, dx_ref, dy_ref, dz_ref, ox_ref, oy_ref, oz_ref):
    s = lax.dot_general(w_ref[:], m_ref[:], (((1,), (1,)), ((), ())),
                        preferred_element_type=jnp.float32)
    ox_ref[0] = dx_ref[0] * s
    oy_ref[0] = dy_ref[0] * s
    oz_ref[0] = dz_ref[0] * s


_noise_w = pl.pallas_call(
    _nw_body,
    grid=(_EG,),
    in_specs=[
        pl.BlockSpec((1, D), lambda i: (0, 0)),
        pl.BlockSpec((_EB, D), lambda i: (i, 0)),
        pl.BlockSpec((1, 1, _EB), lambda i: (i, 0, 0)),
        pl.BlockSpec((1, 1, _EB), lambda i: (i, 0, 0)),
        pl.BlockSpec((1, 1, _EB), lambda i: (i, 0, 0)),
    ],
    out_specs=[
        pl.BlockSpec((1, 1, _EB), lambda i: (i, 0, 0)),
        pl.BlockSpec((1, 1, _EB), lambda i: (i, 0, 0)),
        pl.BlockSpec((1, 1, _EB), lambda i: (i, 0, 0)),
    ],
    out_shape=[
        jax.ShapeDtypeStruct((_EG, 1, _EB), jnp.float32),
        jax.ShapeDtypeStruct((_EG, 1, _EB), jnp.float32),
        jax.ShapeDtypeStruct((_EG, 1, _EB), jnp.float32),
    ],
)


def _nsum_body(p_ref, o_ref):
    o_ref[:] = jnp.sum(p_ref[:], axis=0, keepdims=True)


_noise_sum = pl.pallas_call(
    _nsum_body,
    grid=(1,),
    in_specs=[pl.BlockSpec((NW, 3 * N), lambda i: (0, 0))],
    out_specs=pl.BlockSpec((1, 3 * N), lambda i: (0, 0)),
    out_shape=jax.ShapeDtypeStruct((1, 3 * N), jnp.float32),
)


def kernel(Z, block_emb, edge_emb, W1, b1, W2, b2, w_noise, B, edge_index,
           edge_types):
    f32 = jnp.float32
    zc = Z[:, 0, :].astype(f32)
    src = edge_index[0].astype(jnp.int32)
    dst = edge_index[1].astype(jnp.int32)
    typ = edge_types.astype(jnp.int32)

    dx, dy, dz, d2 = _sc_prep(zc[:, 0], zc[:, 1], zc[:, 2], src, dst)
    d2r = d2.reshape(_EG, 1, _EB)
    typr = typ.reshape(_EG, 1, _EB)
    br = B.astype(jnp.int32).reshape(_NG, 1, _NB)

    h, hw = _embed_kernel(br, block_emb, W1[0, :D])
    for l in range(NLAYERS):
        c = _c_kernel(d2r, typr, W1[l, D + ES:], W1[l, D:D + ES], edge_emb,
                      b1[l:l + 1])
        if l < NLAYERS - 1:
            (aggp,) = _sc_edge(hw, c, src, dst)
            h, hw = _update_next(h, aggp, W2[l], b2[l:l + 1], W1[l + 1, :D])
        else:
            aggp, m = _sc_edge_m(hw, c, src, dst)
            ox, oy, oz = _noise_w(w_noise.T,
                                  m,
                                  dx.reshape(_EG, 1, _EB),
                                  dy.reshape(_EG, 1, _EB),
                                  dz.reshape(_EG, 1, _EB))
            (nzp,) = _sc_nscatter(ox.reshape(E), oy.reshape(E),
                                  oz.reshape(E), dst)
            (h,) = _update_final(h, aggp, W2[l], b2[l:l + 1])

    noise = _noise_sum(nzp)[0].reshape(3, N).T
    return h, noise


# speedup vs baseline: 5.4852x; 1.0363x over previous
import functools

import jax
import jax.numpy as jnp
from jax import lax
from jax.experimental import pallas as pl
from jax.experimental.pallas import tpu as pltpu
from jax.experimental.pallas import tpu_sc as plsc

N = 10000
E = 320000
D = 128
ES = 16
RBF = 16
NLAYERS = 3
NBLK = 100
CUTOFF = 7.0

NC = 2
NS = 16
NW = NC * NS
EPW = E // NW
PCH = 400
NPCH = EPW // PCH
NP = 10240
RPT = NP // NS
VD = D // 16

_mesh = plsc.VectorSubcoreMesh(core_axis_name="c", subcore_axis_name="s")
_sc_params = pltpu.CompilerParams(needs_layout_passes=False)


@functools.partial(
    pl.kernel,
    mesh=_mesh,
    compiler_params=_sc_params,
    out_type=[
        jax.ShapeDtypeStruct((E,), jnp.float32),
        jax.ShapeDtypeStruct((E,), jnp.float32),
        jax.ShapeDtypeStruct((E,), jnp.float32),
        jax.ShapeDtypeStruct((E,), jnp.float32),
    ],
    scratch_types=[
        pltpu.VMEM((N,), jnp.float32),
        pltpu.VMEM((N,), jnp.float32),
        pltpu.VMEM((N,), jnp.float32),
        pltpu.VMEM((PCH,), jnp.int32),
        pltpu.VMEM((PCH,), jnp.int32),
        pltpu.VMEM((PCH,), jnp.float32),
        pltpu.VMEM((PCH,), jnp.float32),
        pltpu.VMEM((PCH,), jnp.float32),
        pltpu.VMEM((PCH,), jnp.float32),
    ],
)
def _sc_prep(zx, zy, zz, srch, dsth, dx_out, dy_out, dz_out, d2_out,
             xv, yv, zv, sv, dv, dxv, dyv, dzv, d2v):
    cid = lax.axis_index("c")
    sid = lax.axis_index("s")
    wid = cid * NS + sid
    pltpu.sync_copy(zx, xv)
    pltpu.sync_copy(zy, yv)
    pltpu.sync_copy(zz, zv)
    base0 = wid * EPW

    def chunk(k, carry):
        base = base0 + k * PCH
        pltpu.sync_copy(srch.at[pl.ds(base, PCH)], sv)
        pltpu.sync_copy(dsth.at[pl.ds(base, PCH)], dv)

        def grp(g, c2):
            si = sv[pl.ds(g * 16, 16)]
            di = dv[pl.ds(g * 16, 16)]
            dx = plsc.load_gather(xv, [di]) - plsc.load_gather(xv, [si])
            dy = plsc.load_gather(yv, [di]) - plsc.load_gather(yv, [si])
            dz = plsc.load_gather(zv, [di]) - plsc.load_gather(zv, [si])
            dxv[pl.ds(g * 16, 16)] = dx
            dyv[pl.ds(g * 16, 16)] = dy
            dzv[pl.ds(g * 16, 16)] = dz
            d2v[pl.ds(g * 16, 16)] = dx * dx + dy * dy + dz * dz
            return c2

        lax.fori_loop(0, PCH // 16, grp, 0)
        pltpu.sync_copy(dxv, dx_out.at[pl.ds(base, PCH)])
        pltpu.sync_copy(dyv, dy_out.at[pl.ds(base, PCH)])
        pltpu.sync_copy(dzv, dz_out.at[pl.ds(base, PCH)])
        pltpu.sync_copy(d2v, d2_out.at[pl.ds(base, PCH)])
        return carry

    lax.fori_loop(0, NPCH, chunk, 0)


def _make_edge(write_m):
    ch = 40 if write_m else 80
    nch = EPW // ch
    pairs = nch // 2
    out_type = [jax.ShapeDtypeStruct((NC, NP, D), jnp.float32)]
    if write_m:
        out_type.append(jax.ShapeDtypeStruct((E, D), jnp.float32))
    scratch = [
        pltpu.VMEM((ch,), jnp.int32),
        pltpu.VMEM((ch,), jnp.int32),
        pltpu.VMEM((ch,), jnp.int32),
        pltpu.VMEM((ch,), jnp.int32),
        pltpu.VMEM((ch, D), jnp.float32),
        pltpu.VMEM((ch, D), jnp.float32),
        pltpu.VMEM((ch, D), jnp.float32),
        pltpu.VMEM((ch, D), jnp.float32),
        pltpu.VMEM_SHARED((NP, D), jnp.float32),
        pltpu.SemaphoreType.DMA,
        pltpu.SemaphoreType.DMA,
    ]
    if write_m:
        scratch += [
            pltpu.VMEM((ch, D), jnp.float32),
            pltpu.VMEM((ch, D), jnp.float32),
            pltpu.VMEM((ch,), jnp.int32),
            pltpu.VMEM((ch,), jnp.int32),
            pltpu.SemaphoreType.DMA,
            pltpu.SemaphoreType.DMA,
        ]

    def body(hw, chbm, srch, dsth, *rest):
        if write_m:
            (agg_out, m_out, sv0, dv0, sv1, dv1, gv0, gv1, cv0, cv1,
             aggsh, sem0, sem1, mv0, mv1, ds0, ds1, semw, sems) = rest
        else:
            (agg_out, sv0, dv0, sv1, dv1, gv0, gv1, cv0, cv1,
             aggsh, sem0, sem1) = rest
            mv0 = gv0
            mv1 = gv1
            ds0 = dv0
            ds1 = dv1
            m_out = semw = sems = None
        del rest
        cid = lax.axis_index("c")
        sid = lax.axis_index("s")
        wid = cid * NS + sid
        zero16 = jnp.zeros((16,), jnp.float32)
        base0 = wid * EPW

        def zfill(j, carry):
            for t in range(VD):
                gv0[j, pl.ds(t * 16, 16)] = zero16
            return carry

        lax.fori_loop(0, ch, zfill, 0)
        for i in range(RPT // ch):
            pltpu.sync_copy(gv0, aggsh.at[pl.ds(sid * RPT + i * ch, ch)])
        plsc.subcore_barrier()

        def issue(k, sv, dv, gv, cv, sem):
            base = base0 + k * ch
            pltpu.sync_copy(srch.at[pl.ds(base, ch)], sv)
            pltpu.sync_copy(dsth.at[pl.ds(base, ch)], dv)
            pltpu.async_copy(hw.at[sv], gv, sem)
            pltpu.async_copy(chbm.at[pl.ds(base, ch)], cv, sem)

        def wait(sv, gv, cv, sem):
            pltpu.make_async_copy(hw.at[sv], gv, sem).wait()
            pltpu.make_async_copy(chbm.at[pl.ds(0, ch)], cv, sem).wait()

        def process(k, dv, gv, cv, mv, dscat, first):
            if write_m and not first:
                pltpu.make_async_copy(mv, m_out.at[pl.ds(0, ch)], semw).wait()
                pltpu.make_async_copy(mv, aggsh.at[dscat], sems).wait()

            def row(j, c2):
                for u in range(4):
                    r = 4 * j + u
                    for t in range(VD):
                        x = gv[r, pl.ds(t * 16, 16)] + cv[r, pl.ds(t * 16, 16)]
                        mv[r, pl.ds(t * 16, 16)] = x / (1.0 + jnp.exp(-x))
                return c2

            lax.fori_loop(0, ch // 4, row, 0)
            if write_m:
                pltpu.async_copy(mv, m_out.at[pl.ds(base0 + k * ch, ch)],
                                 semw)
                for o in (0, 16, ch - 16):
                    dscat[pl.ds(o, 16)] = dv[pl.ds(o, 16)]
                pltpu.async_copy(mv, aggsh.at[dscat], sems, add=True)
            else:
                pltpu.sync_copy(mv, aggsh.at[dv], add=True)

        issue(0, sv0, dv0, gv0, cv0, sem0)
        issue(1, sv1, dv1, gv1, cv1, sem1)
        wait(sv0, gv0, cv0, sem0)
        process(0, dv0, gv0, cv0, mv0, ds0, True)
        issue(2, sv0, dv0, gv0, cv0, sem0)
        wait(sv1, gv1, cv1, sem1)
        process(1, dv1, gv1, cv1, mv1, ds1, True)

        def pipe(i, carry):
            issue(2 * i + 3, sv1, dv1, gv1, cv1, sem1)
            wait(sv0, gv0, cv0, sem0)
            process(2 * i + 2, dv0, gv0, cv0, mv0, ds0, False)
            issue(2 * i + 4, sv0, dv0, gv0, cv0, sem0)
            wait(sv1, gv1, cv1, sem1)
            process(2 * i + 3, dv1, gv1, cv1, mv1, ds1, False)
            return carry

        if nch % 2 == 0:
            lax.fori_loop(0, pairs - 2, pipe, 0)
            issue(nch - 1, sv1, dv1, gv1, cv1, sem1)
            wait(sv0, gv0, cv0, sem0)
            process(nch - 2, dv0, gv0, cv0, mv0, ds0, False)
            wait(sv1, gv1, cv1, sem1)
            process(nch - 1, dv1, gv1, cv1, mv1, ds1, False)
        else:
            lax.fori_loop(0, (nch - 3) // 2, pipe, 0)
            wait(sv0, gv0, cv0, sem0)
            process(nch - 1, dv0, gv0, cv0, mv0, ds0, False)
        if write_m:
            pltpu.make_async_copy(mv0, m_out.at[pl.ds(0, ch)], semw).wait()
            pltpu.make_async_copy(mv1, m_out.at[pl.ds(0, ch)], semw).wait()
            pltpu.make_async_copy(mv0, aggsh.at[ds0], sems).wait()
            pltpu.make_async_copy(mv1, aggsh.at[ds1], sems).wait()

        plsc.subcore_barrier()
        pltpu.sync_copy(aggsh.at[pl.ds(sid * RPT, RPT)],
                        agg_out.at[cid, pl.ds(sid * RPT, RPT)])

    return pl.kernel(body, mesh=_mesh, out_type=out_type,
                     scratch_types=scratch, compiler_params=_sc_params)


_sc_edge = _make_edge(write_m=False)
_sc_edge_m = _make_edge(write_m=True)


_CH2 = 2000
_NCH2 = EPW // _CH2


@functools.partial(
    pl.kernel,
    mesh=_mesh,
    compiler_params=_sc_params,
    out_type=[jax.ShapeDtypeStruct((NW, 3 * N), jnp.float32)],
    scratch_types=[
        pltpu.VMEM((_CH2,), jnp.int32),
        pltpu.VMEM((_CH2,), jnp.float32),
        pltpu.VMEM((_CH2,), jnp.float32),
        pltpu.VMEM((_CH2,), jnp.float32),
        pltpu.VMEM((3 * N,), jnp.float32),
    ],
)
def _sc_nscatter(nxh, nyh, nzh, dsth, nz_out, dv, xv, yv, zv, noise3):
    cid = lax.axis_index("c")
    sid = lax.axis_index("s")
    wid = cid * NS + sid
    zero16 = jnp.zeros((16,), jnp.float32)

    def zfill(i, carry):
        noise3[pl.ds(i * 16, 16)] = zero16
        return carry

    lax.fori_loop(0, 3 * N // 16, zfill, 0)

    lane = lax.broadcasted_iota(jnp.int32, (16,), 0)
    msk3 = lane < 3
    lane3 = jnp.minimum(lane, 2)
    base0 = wid * EPW

    def chunk(k, carry):
        base = base0 + k * _CH2
        pltpu.sync_copy(dsth.at[pl.ds(base, _CH2)], dv)
        pltpu.sync_copy(nxh.at[pl.ds(base, _CH2)], xv)
        pltpu.sync_copy(nyh.at[pl.ds(base, _CH2)], yv)
        pltpu.sync_copy(nzh.at[pl.ds(base, _CH2)], zv)

        def grp(g, c2):
            dgrp = dv[pl.ds(g * 16, 16)]
            xg = xv[pl.ds(g * 16, 16)]
            yg = yv[pl.ds(g * 16, 16)]
            zg = zv[pl.ds(g * 16, 16)]
            for u in range(16):
                val = jnp.where(lane == 0, xg[u],
                                jnp.where(lane == 1, yg[u], zg[u]))
                plsc.addupdate_scatter(noise3, [lane3 * N + dgrp[u]], val,
                                       mask=msk3)
            return c2

        lax.fori_loop(0, _CH2 // 16, grp, 0)
        return carry

    lax.fori_loop(0, _NCH2, chunk, 0)
    pltpu.sync_copy(noise3, nz_out.at[wid])


_EB = 2560
_EG = E // _EB
_NB = 1000
_NG = N // _NB


def _c_body(d2_ref, typ_ref, w1r_ref, w1e_ref, eemb_ref, b1_ref, c_ref):
    d2 = d2_ref[0]
    dist = jnp.sqrt(d2 + 1e-8)
    cen = lax.broadcasted_iota(jnp.int32, (RBF, _EB), 0).astype(jnp.float32) * (
        CUTOFF / (RBF - 1))
    rbf_t = jnp.exp(-(dist - cen) ** 2)
    t = typ_ref[0]
    oh_t = (lax.broadcasted_iota(jnp.int32, (4, _EB), 0) == t).astype(jnp.float32)
    ew = jnp.dot(eemb_ref[:], w1e_ref[:],
                 preferred_element_type=jnp.float32)
    c = lax.dot_general(rbf_t, w1r_ref[:], (((0,), (0,)), ((), ())),
                        preferred_element_type=jnp.float32)
    c += lax.dot_general(oh_t, ew, (((0,), (0,)), ((), ())),
                         preferred_element_type=jnp.float32)
    c_ref[:] = c + b1_ref[:]


_c_kernel = pl.pallas_call(
    _c_body,
    grid=(_EG,),
    in_specs=[
        pl.BlockSpec((1, 1, _EB), lambda i: (i, 0, 0)),
        pl.BlockSpec((1, 1, _EB), lambda i: (i, 0, 0)),
        pl.BlockSpec((RBF, D), lambda i: (0, 0)),
        pl.BlockSpec((ES, D), lambda i: (0, 0)),
        pl.BlockSpec((4, ES), lambda i: (0, 0)),
        pl.BlockSpec((1, D), lambda i: (0, 0)),
    ],
    out_specs=pl.BlockSpec((_EB, D), lambda i: (i, 0)),
    out_shape=jax.ShapeDtypeStruct((E, D), jnp.float32),
)


def _embed_body(b_ref, emb_ref, w1h_ref, h_ref, hw_ref):
    b = b_ref[0]
    oh_t = (lax.broadcasted_iota(jnp.int32, (NBLK, _NB), 0) == b).astype(jnp.float32)
    h = lax.dot_general(oh_t, emb_ref[:], (((0,), (0,)), ((), ())),
                        preferred_element_type=jnp.float32)
    h_ref[:] = h
    hw_ref[:] = jnp.dot(h, w1h_ref[:], preferred_element_type=jnp.float32)


_embed_kernel = pl.pallas_call(
    _embed_body,
    grid=(_NG,),
    in_specs=[
        pl.BlockSpec((1, 1, _NB), lambda i: (i, 0, 0)),
        pl.BlockSpec((NBLK, D), lambda i: (0, 0)),
        pl.BlockSpec((D, D), lambda i: (0, 0)),
    ],
    out_specs=[
        pl.BlockSpec((_NB, D), lambda i: (i, 0)),
        pl.BlockSpec((_NB, D), lambda i: (i, 0)),
    ],
    out_shape=[
        jax.ShapeDtypeStruct((N, D), jnp.float32),
        jax.ShapeDtypeStruct((N, D), jnp.float32),
    ],
)


def _make_update(with_next):
    def body(h_ref, aggp_ref, w2_ref, b2_ref, *rest):
        agg = aggp_ref[0] + aggp_ref[1]
        u = jnp.dot(agg, w2_ref[:], preferred_element_type=jnp.float32)
        u = u + b2_ref[:]
        hn = h_ref[:] + u * (1.0 / (1.0 + jnp.exp(-u)))
        if with_next:
            w1n_ref, hn_ref, hw_ref = rest
            hn_ref[:] = hn
            hw_ref[:] = jnp.dot(hn, w1n_ref[:],
                                preferred_element_type=jnp.float32)
        else:
            (hn_ref,) = rest
            hn_ref[:] = hn

    in_specs = [
        pl.BlockSpec((_NB, D), lambda i: (i, 0)),
        pl.BlockSpec((NC, _NB, D), lambda i: (0, i, 0)),
        pl.BlockSpec((D, D), lambda i: (0, 0)),
        pl.BlockSpec((1, D), lambda i: (0, 0)),
    ]
    out_specs = [pl.BlockSpec((_NB, D), lambda i: (i, 0))]
    out_shape = [jax.ShapeDtypeStruct((N, D), jnp.float32)]
    if with_next:
        in_specs.append(pl.BlockSpec((D, D), lambda i: (0, 0)))
        out_specs.append(pl.BlockSpec((_NB, D), lambda i: (i, 0)))
        out_shape.append(jax.ShapeDtypeStruct((N, D), jnp.float32))
    return pl.pallas_call(body, grid=(_NG,), in_specs=in_specs,
                          out_specs=out_specs, out_shape=out_shape)


_update_next = _make_update(True)
_update_final = _make_update(False)


def _nw_body(w_ref, m_ref, dx_ref, dy_ref, dz_ref, ox_ref, oy_ref, oz_ref):
    s = lax.dot_general(w_ref[:], m_ref[:], (((1,), (1,)), ((), ())),
                        preferred_element_type=jnp.float32)
    ox_ref[0] = dx_ref[0] * s
    oy_ref[0] = dy_ref[0] * s
    oz_ref[0] = dz_ref[0] * s


_noise_w = pl.pallas_call(
    _nw_body,
    grid=(_EG,),
    in_specs=[
        pl.BlockSpec((1, D), lambda i: (0, 0)),
        pl.BlockSpec((_EB, D), lambda i: (i, 0)),
        pl.BlockSpec((1, 1, _EB), lambda i: (i, 0, 0)),
        pl.BlockSpec((1, 1, _EB), lambda i: (i, 0, 0)),
        pl.BlockSpec((1, 1, _EB), lambda i: (i, 0, 0)),
    ],
    out_specs=[
        pl.BlockSpec((1, 1, _EB), lambda i: (i, 0, 0)),
        pl.BlockSpec((1, 1, _EB), lambda i: (i, 0, 0)),
        pl.BlockSpec((1, 1, _EB), lambda i: (i, 0, 0)),
    ],
    out_shape=[
        jax.ShapeDtypeStruct((_EG, 1, _EB), jnp.float32),
        jax.ShapeDtypeStruct((_EG, 1, _EB), jnp.float32),
        jax.ShapeDtypeStruct((_EG, 1, _EB), jnp.float32),
    ],
)


def _nsum_body(p_ref, o_ref):
    o_ref[:] = jnp.sum(p_ref[:], axis=0, keepdims=True)


_noise_sum = pl.pallas_call(
    _nsum_body,
    grid=(1,),
    in_specs=[pl.BlockSpec((NW, 3 * N), lambda i: (0, 0))],
    out_specs=pl.BlockSpec((1, 3 * N), lambda i: (0, 0)),
    out_shape=jax.ShapeDtypeStruct((1, 3 * N), jnp.float32),
)


def kernel(Z, block_emb, edge_emb, W1, b1, W2, b2, w_noise, B, edge_index,
           edge_types):
    f32 = jnp.float32
    zc = Z[:, 0, :].astype(f32)
    src = edge_index[0].astype(jnp.int32)
    dst = edge_index[1].astype(jnp.int32)
    typ = edge_types.astype(jnp.int32)

    dx, dy, dz, d2 = _sc_prep(zc[:, 0], zc[:, 1], zc[:, 2], src, dst)
    d2r = d2.reshape(_EG, 1, _EB)
    typr = typ.reshape(_EG, 1, _EB)
    br = B.astype(jnp.int32).reshape(_NG, 1, _NB)

    h, hw = _embed_kernel(br, block_emb, W1[0, :D])
    for l in range(NLAYERS):
        c = _c_kernel(d2r, typr, W1[l, D + ES:], W1[l, D:D + ES], edge_emb,
                      b1[l:l + 1])
        if l < NLAYERS - 1:
            (aggp,) = _sc_edge(hw, c, src, dst)
            h, hw = _update_next(h, aggp, W2[l], b2[l:l + 1], W1[l + 1, :D])
        else:
            aggp, m = _sc_edge_m(hw, c, src, dst)
            ox, oy, oz = _noise_w(w_noise.T,
                                  m,
                                  dx.reshape(_EG, 1, _EB),
                                  dy.reshape(_EG, 1, _EB),
                                  dz.reshape(_EG, 1, _EB))
            (nzp,) = _sc_nscatter(ox.reshape(E), oy.reshape(E),
                                  oz.reshape(E), dst)
            (h,) = _update_final(h, aggp, W2[l], b2[l:l + 1])

    noise = _noise_sum(nzp)[0].reshape(3, N).T
    return h, noise
